# Initial kernel scaffold; baseline (speedup 1.0000x reference)
import jax
import jax.numpy as jnp
from jax.experimental import pallas as pl


def kernel(x_protein, hyperedge_protein_index, x_meta, hyperedge_meta_index, index, params):
    raise NotImplementedError("write your pallas kernel here")


# speedup vs baseline: 3.9066x; 3.9066x over previous
import functools

import jax
import jax.numpy as jnp
import numpy as np
from jax import lax
from jax.experimental import pallas as pl
from jax.experimental.pallas import tpu as pltpu
from jax.experimental.pallas import tpu_sc as plsc

N = 10000
NP = 10240
E = 160000
BN_EPS = 1e-5
NTILES = 16
NCORES = 2
EPT = E // NTILES
CHUNK = 125
NCHUNK = EPT // CHUNK
RPT = NP // NTILES
WCH = 128
NWCH = RPT // WCH
SWIN = 656

def _tile_windows(inv):
    ext = jnp.concatenate([inv, inv[:, -16:]], axis=1)
    return jnp.stack([ext[:, RPT * s:RPT * s + SWIN] for s in range(NTILES)],
                     axis=1)


@functools.lru_cache(maxsize=None)
def _mesh():
    return plsc.VectorSubcoreMesh(core_axis_name="c", subcore_axis_name="s")


@functools.lru_cache(maxsize=None)
def _make_degrees():
    return functools.partial(
        pl.kernel,
        mesh=_mesh(),
        out_type=jax.ShapeDtypeStruct((8, NP), jnp.float32),
        scratch_types=[
            pltpu.VMEM((NCHUNK, CHUNK), jnp.int32),
            pltpu.VMEM((CHUNK,), jnp.float32),
            pltpu.VMEM((RPT,), jnp.float32),
            pltpu.VMEM_SHARED((NP,), jnp.float32),
            pltpu.SemaphoreType.DMA,
        ],
    )(_sc_degrees_body)


def _sc_degrees_body(idx_hbm, ones_hbm, zeros_hbm, inv_hbm,
                     idx_v, ones_v, val_v, acc, sem):
    c = lax.axis_index("c")
    s = lax.axis_index("s")
    pltpu.sync_copy(ones_hbm, ones_v)
    for jb in range(4):
        job = c * 4 + jb
        pltpu.sync_copy(zeros_hbm, acc.at[pl.ds(s * RPT, RPT)])
        pltpu.sync_copy(idx_hbm.at[job, s], idx_v)
        plsc.subcore_barrier()

        def grp(g, _):
            descs = []
            for u in range(8):
                descs.append(pltpu.async_copy(
                    ones_v, acc.at[idx_v.at[g * 8 + u]], sem, add=True))
            for d in descs:
                d.wait()
            return ()

        lax.fori_loop(0, NCHUNK // 8, grp, ())
        plsc.subcore_barrier()
        pltpu.sync_copy(acc.at[pl.ds(s * RPT, RPT)], val_v)
        for l in range(RPT // 16):
            sl = pl.ds(l * 16, 16)
            v = val_v[sl]
            val_v[sl] = jnp.where(v > 0.0, 1.0 / v, 0.0)
        pltpu.sync_copy(val_v, inv_hbm.at[job, pl.ds(s * RPT, RPT)])
        plsc.subcore_barrier()


def _sc_degrees(idx8):
    ones = jnp.ones((CHUNK,), jnp.float32)
    zeros = jnp.zeros((RPT,), jnp.float32)
    return _make_degrees()(idx8, ones, zeros)


DH2 = 128
SUP = 16
NSUP = NCHUNK // SUP
WROWS = 64
NWO = RPT // WROWS


@functools.lru_cache(maxsize=None)
def _make_hconv():
    out_sd = jax.ShapeDtypeStruct((NCORES * NP, DH2), jnp.float32)

    @functools.partial(
        pl.kernel,
        mesh=_mesh(),
        out_type=(out_sd, out_sd),
        scratch_types=[
            pltpu.VMEM((SUP, CHUNK), jnp.int32),
            pltpu.VMEM((SUP, CHUNK), jnp.int32),
            pltpu.VMEM((CHUNK, DH2), jnp.float32),
            pltpu.VMEM((CHUNK, DH2), jnp.float32),
            pltpu.VMEM((SWIN,), jnp.float32),
            pltpu.VMEM((DH2,), jnp.float32),
            pltpu.VMEM_SHARED((NP, DH2), jnp.float32),
            pltpu.SemaphoreType.DMA,
            pltpu.SemaphoreType.DMA,
        ],
    )
    def hconv(table, src1, dst1, src2, dst2, scales, biases, zeros,
              out_final, out_ef,
              src_v, dst_v, buf0, buf1, scale_v, bias_v, acc,
              gsem, ssem):
        c = lax.axis_index("c")
        s = lax.axis_index("s")
        bufs = (buf0, buf1)
        for p in range(2):
            src = src1 if p == 0 else src2
            dst = dst1 if p == 0 else dst2
            tab = table if p == 0 else out_ef
            out = out_ef if p == 0 else out_final
            pltpu.sync_copy(zeros, acc.at[pl.ds(s * RPT, RPT)])
            pltpu.sync_copy(scales.at[p, s], scale_v)
            pltpu.sync_copy(biases.at[p, c], bias_v)
            plsc.subcore_barrier()

            for si in range(NSUP):
                pltpu.sync_copy(src.at[c, s, pl.ds(si * SUP, SUP)], src_v)
                pltpu.sync_copy(dst.at[s, pl.ds(si * SUP, SUP)], dst_v)
                pltpu.async_copy(tab.at[src_v.at[0]], buf0, gsem)
                pltpu.async_copy(tab.at[src_v.at[1]], buf1, gsem)

                def step(j2, _):
                    for b in range(2):
                        j = j2 * 2 + b
                        buf = bufs[b]
                        pltpu.make_async_copy(
                            tab.at[src_v.at[j]], buf, gsem).wait()
                        pltpu.async_copy(
                            buf, acc.at[dst_v.at[j]], ssem, add=True).wait()

                        @pl.when(j + 2 < SUP)
                        def _():
                            pltpu.async_copy(
                                tab.at[src_v.at[j + 2]], buf, gsem)
                    return ()

                lax.fori_loop(0, SUP // 2, step, ())
            plsc.subcore_barrier()

            for k in range(NWO):
                row0 = s * RPT + k * WROWS
                wbuf = buf0.at[pl.ds(0, WROWS)]
                pltpu.sync_copy(acc.at[pl.ds(row0, WROWS)], wbuf)

                def rowfix(r, _, _k=k):
                    sc = scale_v[pl.ds(_k * WROWS + r, 16)][0]
                    for l in range(DH2 // 16):
                        sl = pl.ds(l * 16, 16)
                        buf0[r, sl] = buf0[r, sl] * sc + bias_v[sl]
                    return ()

                lax.fori_loop(0, WROWS, rowfix, ())
                pltpu.sync_copy(wbuf, out.at[pl.ds(c * NP + row0, WROWS)])
            plsc.subcore_barrier()

    return hconv


def _sc_hconv(table, plan, scales, biases):
    zeros = jnp.zeros((RPT, DH2), jnp.float32)
    final, _ef = _make_hconv()(
        table, plan["src1"], plan["dst1"], plan["src2"], plan["dst2"],
        scales, biases, zeros)
    return final


@functools.lru_cache(maxsize=None)
def _make_gather():
    return functools.partial(
        pl.kernel,
        mesh=_mesh(),
        out_type=jax.ShapeDtypeStruct((8192, 128), jnp.float32),
        scratch_types=[
            pltpu.VMEM((2, 128), jnp.int32),
            pltpu.VMEM((128, 128), jnp.float32),
            pltpu.VMEM((128, 128), jnp.float32),
            pltpu.SemaphoreType.DMA,
        ],
    )(_sc_gather_body)


def _sc_gather(tables, idx_all):
    return _make_gather()(tables, idx_all)


def _sc_gather_body(tab, idx, out, idx_v, bufa, bufb, sem):
    c = lax.axis_index("c")
    s = lax.axis_index("s")
    w = s * NCORES + c
    pltpu.sync_copy(idx.at[w], idx_v)
    da = pltpu.async_copy(tab.at[idx_v.at[0]], bufa, sem)
    db = pltpu.async_copy(tab.at[idx_v.at[1]], bufb, sem)
    da.wait()
    pltpu.sync_copy(bufa, out.at[pl.ds(w * 256, 128)])
    db.wait()
    pltpu.sync_copy(bufb, out.at[pl.ds(w * 256 + 128, 128)])


def _dense_body(x_ref, w_ref, b_ref, o_ref):
    kc = x_ref.shape[0]
    ns = w_ref.shape[2]
    for h in range(ns):
        acc = jnp.dot(x_ref[0], w_ref[0, :, h],
                      preferred_element_type=jnp.float32)
        for k in range(1, kc):
            acc += jnp.dot(x_ref[k], w_ref[k, :, h],
                           preferred_element_type=jnp.float32)
        o_ref[h] = acc + b_ref[h]


def _dense(x, w, b, br=2048):
    kc, n, k2 = x.shape
    ns, nh2 = w.shape[2], w.shape[3]
    grid = (n // br,)
    return pl.pallas_call(
        _dense_body,
        grid=grid,
        in_specs=[
            pl.BlockSpec((kc, br, k2), lambda i: (0, i, 0)),
            pl.BlockSpec((kc, k2, ns, nh2), lambda i: (0, 0, 0, 0)),
            pl.BlockSpec((ns, 1, nh2), lambda i: (0, 0, 0)),
        ],
        out_specs=pl.BlockSpec((ns, br, nh2), lambda i: (0, i, 0)),
        out_shape=jax.ShapeDtypeStruct((ns, n, nh2), jnp.float32),
    )(x, w, b)


_CBI = 1024
_CNB = NP // _CBI


def _contrast_body(a_ref, b_ref, d_ref, lr_ref, lc_ref, rowacc, colacc):
    i = pl.program_id(0)
    j = pl.program_id(1)
    a = a_ref[...]
    b = b_ref[...]
    a = a * lax.rsqrt(jnp.maximum(jnp.sum(a * a, 1, keepdims=True), 1e-24))
    b = b * lax.rsqrt(jnp.maximum(jnp.sum(b * b, 1, keepdims=True), 1e-24))
    s = lax.dot_general(a, b, (((1,), (1,)), ((), ())),
                        preferred_element_type=jnp.float32) * (1.0 / 0.7)
    rid = i * _CBI + lax.broadcasted_iota(jnp.int32, (_CBI, _CBI), 0)
    cid = j * _CBI + lax.broadcasted_iota(jnp.int32, (_CBI, _CBI), 1)
    ex = jnp.where(cid < N, jnp.exp(s), 0.0)
    rs_ = jnp.sum(ex, axis=1, keepdims=True)
    cs_ = jnp.sum(ex * jnp.where(rid < N, 1.0, 0.0), axis=0, keepdims=True)
    zz = jnp.zeros((1, 1), jnp.float32)

    @pl.when((i == 0) & (j == 0))
    def _():
        d_ref[...] = zz
        lr_ref[...] = zz
        lc_ref[...] = zz

    @pl.when(j == 0)
    def _():
        rowacc[...] = rs_

    @pl.when(j > 0)
    def _():
        rowacc[...] += rs_

    @pl.when(i == 0)
    def _():
        colacc[j] = cs_

    @pl.when(i > 0)
    def _():
        colacc[j] += cs_

    @pl.when(i == j)
    def _():
        d_ref[...] += jnp.sum(
            jnp.where((rid == cid) & (rid < N), s, 0.0)).reshape(1, 1)

    @pl.when(j == _CNB - 1)
    def _():
        rvid = i * _CBI + lax.broadcasted_iota(jnp.int32, (_CBI, 1), 0)
        lr_ref[...] += jnp.sum(
            jnp.where(rvid < N, jnp.log(rowacc[...]), 0.0)).reshape(1, 1)

    @pl.when(i == _CNB - 1)
    def _():
        cvid = j * _CBI + lax.broadcasted_iota(jnp.int32, (1, _CBI), 1)
        lc_ref[...] += jnp.sum(
            jnp.where(cvid < N, jnp.log(colacc[j]), 0.0)).reshape(1, 1)


def _contrast_tc(a, b):
    sd = jax.ShapeDtypeStruct((1, 1), jnp.float32)
    d, lr, lc = pl.pallas_call(
        _contrast_body,
        grid=(_CNB, _CNB),
        in_specs=[
            pl.BlockSpec((_CBI, 128), lambda i, j: (i, 0)),
            pl.BlockSpec((_CBI, 128), lambda i, j: (j, 0)),
        ],
        out_specs=[pl.BlockSpec((1, 1), lambda i, j: (0, 0))] * 3,
        out_shape=[sd, sd, sd],
        scratch_shapes=[
            pltpu.VMEM((_CBI, 1), jnp.float32),
            pltpu.VMEM((_CNB, 1, _CBI), jnp.float32),
        ],
    )(a, b)
    return -d[0, 0] / N + (lr[0, 0] + lc[0, 0]) / (2.0 * N)


def _rowatt_body(c0_ref, c1_ref, x_ref, p_ref, o_ref):
    p3 = c0_ref[...]
    p2 = c1_ref[...]
    avg0 = jnp.mean(p3, axis=1, keepdims=True)
    avg1 = jnp.mean(p2, axis=1, keepdims=True)
    aa = jnp.maximum(avg0 * p_ref[0, 0] + avg1 * p_ref[0, 1] + p_ref[0, 2], 0.0)
    at0 = jax.nn.sigmoid(aa * p_ref[0, 3] + p_ref[0, 5])
    at1 = jax.nn.sigmoid(aa * p_ref[0, 4] + p_ref[0, 6])
    o_ref[...] = (p3 * (at0 * p_ref[0, 7]) + p2 * (at1 * p_ref[0, 8])
                  + p_ref[0, 9] + x_ref[...])


def _rowatt(c0, c1, x, prm, br=2048):
    return pl.pallas_call(
        _rowatt_body,
        grid=(NP // br,),
        in_specs=[
            pl.BlockSpec((br, 128), lambda i: (i, 0)),
            pl.BlockSpec((br, 128), lambda i: (i, 0)),
            pl.BlockSpec((br, 128), lambda i: (i, 0)),
            pl.BlockSpec((1, 128), lambda i: (0, 0)),
        ],
        out_specs=pl.BlockSpec((br, 128), lambda i: (i, 0)),
        out_shape=jax.ShapeDtypeStruct((NP, 128), jnp.float32),
    )(c0, c1, x, prm)


def _mlp_body(g_ref, w1_ref, b1_ref, w2_ref, b2_ref, w3_ref, b3_ref, o_ref):
    h = (jnp.dot(g_ref[0], w1_ref[0], preferred_element_type=jnp.float32)
         + jnp.dot(g_ref[1], w1_ref[1], preferred_element_type=jnp.float32)
         + b1_ref[...])
    h = jnp.where(h > 0, h, 0.01 * h)
    h = jnp.dot(h, w2_ref[...], preferred_element_type=jnp.float32) + b2_ref[...]
    h = jnp.where(h > 0, h, 0.01 * h)
    o_ref[...] = (jnp.dot(h, w3_ref[...], preferred_element_type=jnp.float32)
                  + b3_ref[...])


def _mlp(g, w1, b1, w2, b2, w3, b3, br=1024):
    nb = 4096 // br
    return pl.pallas_call(
        _mlp_body,
        grid=(nb,),
        in_specs=[
            pl.BlockSpec((2, br, 128), lambda i: (0, i, 0)),
            pl.BlockSpec((2, 128, 64), lambda i: (0, 0, 0)),
            pl.BlockSpec((1, 64), lambda i: (0, 0)),
            pl.BlockSpec((64, 32), lambda i: (0, 0)),
            pl.BlockSpec((1, 32), lambda i: (0, 0)),
            pl.BlockSpec((32, 1), lambda i: (0, 0)),
            pl.BlockSpec((1, 1), lambda i: (0, 0)),
        ],
        out_specs=pl.BlockSpec((br, 1), lambda i: (i, 0)),
        out_shape=jax.ShapeDtypeStruct((4096, 1), jnp.float32),
    )(g, w1, b1, w2, b2, w3, b3)


def _tiled(x):
    return x.reshape(NTILES, NCHUNK, CHUNK)


def _he_plan(he):
    ni, ei = he[0], he[1]
    return {
        "src1": jnp.stack([_tiled(ni), _tiled(ni + NP)], axis=0),
        "dst1": _tiled(ei),
        "src2": jnp.stack([_tiled(ei), _tiled(ei + NP)], axis=0),
        "dst2": _tiled(ni),
    }


_BN_S = 1.0 / np.sqrt(1.0 + BN_EPS)


def _run_hgnn(x_pad, plan, pp, binv_win, dinv_win):
    scales = jnp.stack([binv_win, dinv_win], axis=0)
    w1 = pp["W1"].reshape(1, 128, 2, 128)
    b0 = jnp.zeros((2, 1, 128), jnp.float32)
    xl1 = _dense(x_pad.reshape(1, NP, 128), w1, b0)
    biases1 = jnp.stack([jnp.zeros((2, 128), jnp.float32),
                         pp["b1"].reshape(2, 128)], axis=0)
    h1 = _sc_hconv(xl1.reshape(2 * NP, 128), plan, scales, biases1)
    s1 = pp["g1"] * _BN_S
    w2p = (s1[:, None] * pp["W2"]).reshape(2, 128, 128)
    w2d = jnp.broadcast_to(w2p[:, :, None, :], (2, 128, 2, 128))
    b2p = pp["be1"] @ pp["W2"]
    b2f = jnp.broadcast_to(b2p[None, None, :], (2, 1, 128))
    xl2 = _dense(h1.reshape(2, NP, 128), w2d, b2f)
    biases2 = jnp.stack([jnp.zeros((2, 128), jnp.float32),
                         jnp.stack([pp["b2"], pp["b2"]])], axis=0)
    h2 = _sc_hconv(xl2.reshape(2 * NP, 128), plan, scales, biases2)
    return h2


def _att_prm(ap):
    v = jnp.stack([ap["fc1W"][0, 0], ap["fc1W"][1, 0], ap["fc1b"][0],
                   ap["fc2W"][0, 0], ap["fc2W"][0, 1],
                   ap["fc2b"][0], ap["fc2b"][1],
                   ap["convW"][0], ap["convW"][1], ap["convb"][0]])
    return jnp.zeros((1, 128), jnp.float32).at[0, :10].set(v)


def kernel(x_protein, hyperedge_protein_index, x_meta, hyperedge_meta_index,
           index, params):
    he_p, he_m = hyperedge_protein_index, hyperedge_meta_index
    idx8 = jnp.stack([
        _tiled(he_p[0, 0]), _tiled(he_p[0, 1]),
        _tiled(he_p[1, 0]), _tiled(he_p[1, 1]),
        _tiled(he_m[0, 0]), _tiled(he_m[0, 1]),
        _tiled(he_m[1, 0]), _tiled(he_m[1, 1]),
    ], axis=0)
    inv = _sc_degrees(idx8)
    win = _tile_windows(inv)

    xp_pad = jnp.pad(x_protein, ((0, NP - N), (0, 0)))
    xm_pad = jnp.pad(x_meta, ((0, NP - N), (0, 0)))

    plans = [_he_plan(he_p[0]), _he_plan(he_p[1]),
             _he_plan(he_m[0]), _he_plan(he_m[1])]
    p2 = _run_hgnn(xp_pad, plans[0], params["pc1"], win[1], win[0])
    p3 = _run_hgnn(xp_pad, plans[1], params["pc2"], win[3], win[2])
    m2 = _run_hgnn(xm_pad, plans[2], params["mc1"], win[5], win[4])
    m3 = _run_hgnn(xm_pad, plans[3], params["mc2"], win[7], win[6])

    loss_p = _contrast_tc(p2, p3)
    loss_m = _contrast_tc(m2, m3)
    loss = (jnp.exp(-params["p12"]) * loss_p + params["p12"]
            + jnp.exp(-params["m13"]) * loss_m + params["m13"])

    protein = _rowatt(p3, p2, xp_pad, _att_prm(params["attp"]))
    meta = _rowatt(m3, m2, xm_pad, _att_prm(params["attm"]))

    tables = jnp.concatenate([protein, meta], axis=0)
    idx_all = jnp.concatenate([index[0], index[1] + NP]).reshape(32, 2, 128)
    g = _sc_gather(tables, idx_all)

    e = params["enc"]
    s1 = e["g1"] * _BN_S
    w1 = (e["W1"] * s1[None, :]).reshape(2, 128, 64)
    b1 = (e["b1"] * s1 + e["be1"]).reshape(1, 64)
    s2 = e["g2"] * _BN_S
    w2 = e["W2"] * s2[None, :]
    b2 = (e["b2"] * s2 + e["be2"]).reshape(1, 32)
    out = _mlp(g.reshape(2, 4096, 128), w1, b1, w2, b2,
               e["W3"], e["b3"].reshape(1, 1))
    return out, loss


# speedup vs baseline: 5.1428x; 1.3165x over previous
import functools

import jax
import jax.numpy as jnp
import numpy as np
from jax import lax
from jax.experimental import pallas as pl
from jax.experimental.pallas import tpu as pltpu
from jax.experimental.pallas import tpu_sc as plsc

N = 10000
NP = 10240
E = 160000
BN_EPS = 1e-5
NTILES = 16
NCORES = 2
CHUNK = 125
NCHUNK = E // NTILES // CHUNK
ECH = E // NCORES // NTILES // CHUNK
RPT = NP // NTILES
DH2 = 128


@functools.lru_cache(maxsize=None)
def _mesh():
    return plsc.VectorSubcoreMesh(core_axis_name="c", subcore_axis_name="s")


@functools.lru_cache(maxsize=None)
def _make_degrees():
    return functools.partial(
        pl.kernel,
        mesh=_mesh(),
        out_type=jax.ShapeDtypeStruct((8, NP), jnp.float32),
        scratch_types=[
            pltpu.VMEM((NCHUNK, CHUNK), jnp.int32),
            pltpu.VMEM((CHUNK,), jnp.float32),
            pltpu.VMEM((RPT,), jnp.float32),
            pltpu.VMEM_SHARED((NP,), jnp.float32),
            pltpu.SemaphoreType.DMA,
        ],
    )(_sc_degrees_body)


def _sc_degrees_body(idx_hbm, ones_hbm, zeros_hbm, inv_hbm,
                     idx_v, ones_v, val_v, acc, sem):
    c = lax.axis_index("c")
    s = lax.axis_index("s")
    pltpu.sync_copy(ones_hbm, ones_v)
    for jb in range(4):
        job = c * 4 + jb
        pltpu.sync_copy(zeros_hbm, acc.at[pl.ds(s * RPT, RPT)])
        pltpu.sync_copy(idx_hbm.at[job, s], idx_v)
        plsc.subcore_barrier()

        def grp(g, _):
            descs = []
            for u in range(8):
                descs.append(pltpu.async_copy(
                    ones_v, acc.at[idx_v.at[g * 8 + u]], sem, add=True))
            for d in descs:
                d.wait()
            return ()

        lax.fori_loop(0, NCHUNK // 8, grp, ())
        plsc.subcore_barrier()
        pltpu.sync_copy(acc.at[pl.ds(s * RPT, RPT)], val_v)
        for l in range(RPT // 16):
            sl = pl.ds(l * 16, 16)
            v = val_v[sl]
            val_v[sl] = jnp.where(v > 0.0, 1.0 / v, 0.0)
        pltpu.sync_copy(val_v, inv_hbm.at[job, pl.ds(s * RPT, RPT)])
        plsc.subcore_barrier()


def _sc_degrees(idx8):
    ones = jnp.ones((CHUNK,), jnp.float32)
    zeros = jnp.zeros((RPT,), jnp.float32)
    return _make_degrees()(idx8, ones, zeros)


@functools.lru_cache(maxsize=None)
def _make_seghalf():
    @functools.partial(
        pl.kernel,
        mesh=_mesh(),
        out_type=jax.ShapeDtypeStruct((NCORES * NP, DH2), jnp.float32),
        scratch_types=[
            pltpu.VMEM((ECH, CHUNK), jnp.int32),
            pltpu.VMEM((ECH, CHUNK), jnp.int32),
            pltpu.VMEM((CHUNK, DH2), jnp.float32),
            pltpu.VMEM((CHUNK, DH2), jnp.float32),
            pltpu.VMEM_SHARED((NP, DH2), jnp.float32),
            pltpu.SemaphoreType.DMA,
            pltpu.SemaphoreType.DMA,
        ],
    )
    def seghalf(table, src, dst, zeros, out,
                src_v, dst_v, buf0, buf1, acc, gsem, ssem):
        c = lax.axis_index("c")
        s = lax.axis_index("s")
        bufs = (buf0, buf1)
        pltpu.sync_copy(zeros, acc.at[pl.ds(s * RPT, RPT)])
        pltpu.sync_copy(src.at[c, s], src_v)
        pltpu.sync_copy(dst.at[c, s], dst_v)
        plsc.subcore_barrier()

        pltpu.async_copy(table.at[src_v.at[0]], buf0, gsem)
        pltpu.async_copy(table.at[src_v.at[1]], buf1, gsem)

        def step(j2, _):
            for b in range(2):
                j = j2 * 2 + b
                buf = bufs[b]
                pltpu.make_async_copy(table.at[src_v.at[j]], buf, gsem).wait()
                pltpu.async_copy(
                    buf, acc.at[dst_v.at[j]], ssem, add=True).wait()

                @pl.when(j + 2 < ECH)
                def _():
                    pltpu.async_copy(table.at[src_v.at[j + 2]], buf, gsem)
            return ()

        lax.fori_loop(0, ECH // 2, step, ())
        plsc.subcore_barrier()
        pltpu.sync_copy(acc.at[pl.ds(s * RPT, RPT)],
                        out.at[pl.ds(c * NP + s * RPT, RPT)])

    return seghalf


def _sc_seghalf(table, src, dst):
    zeros = jnp.zeros((RPT, DH2), jnp.float32)
    return _make_seghalf()(table, src, dst, zeros)


@functools.lru_cache(maxsize=None)
def _make_gather():
    return functools.partial(
        pl.kernel,
        mesh=_mesh(),
        out_type=jax.ShapeDtypeStruct((8192, 128), jnp.float32),
        scratch_types=[
            pltpu.VMEM((2, 128), jnp.int32),
            pltpu.VMEM((128, 128), jnp.float32),
            pltpu.VMEM((128, 128), jnp.float32),
            pltpu.SemaphoreType.DMA,
        ],
    )(_sc_gather_body)


def _sc_gather(tables, idx_all):
    return _make_gather()(tables, idx_all)


def _sc_gather_body(tab, idx, out, idx_v, bufa, bufb, sem):
    c = lax.axis_index("c")
    s = lax.axis_index("s")
    w = s * NCORES + c
    pltpu.sync_copy(idx.at[w], idx_v)
    da = pltpu.async_copy(tab.at[idx_v.at[0]], bufa, sem)
    db = pltpu.async_copy(tab.at[idx_v.at[1]], bufb, sem)
    da.wait()
    pltpu.sync_copy(bufa, out.at[pl.ds(w * 256, 128)])
    db.wait()
    pltpu.sync_copy(bufb, out.at[pl.ds(w * 256 + 128, 128)])


def _merge_body(p_ref, rs_ref, b_ref, o_ref):
    o_ref[...] = (p_ref[0] + p_ref[1]) * rs_ref[...] + b_ref[...]


def _merge(parts, rowscale, bias, br=2048):
    return pl.pallas_call(
        _merge_body,
        grid=(NP // br,),
        in_specs=[
            pl.BlockSpec((2, br, 128), lambda i: (0, i, 0)),
            pl.BlockSpec((br, 1), lambda i: (i, 0)),
            pl.BlockSpec((1, 128), lambda i: (0, 0)),
        ],
        out_specs=pl.BlockSpec((br, 128), lambda i: (i, 0)),
        out_shape=jax.ShapeDtypeStruct((NP, 128), jnp.float32),
    )(parts, rowscale, bias)


def _dense(xs, w, b, br=2048):
    kc = len(xs)
    n, k2 = xs[0].shape
    ns, nh2 = w.shape[2], w.shape[3]

    def body(*refs):
        x_refs = refs[:kc]
        w_ref, b_ref, o_ref = refs[kc], refs[kc + 1], refs[kc + 2]
        for h in range(ns):
            acc = jnp.dot(x_refs[0][...], w_ref[0, :, h],
                          preferred_element_type=jnp.float32)
            for k in range(1, kc):
                acc += jnp.dot(x_refs[k][...], w_ref[k, :, h],
                               preferred_element_type=jnp.float32)
            o_ref[h] = acc + b_ref[h]

    return pl.pallas_call(
        body,
        grid=(n // br,),
        in_specs=[pl.BlockSpec((br, k2), lambda i: (i, 0))] * kc + [
            pl.BlockSpec((kc, k2, ns, nh2), lambda i: (0, 0, 0, 0)),
            pl.BlockSpec((ns, 1, nh2), lambda i: (0, 0, 0)),
        ],
        out_specs=pl.BlockSpec((ns, br, nh2), lambda i: (0, i, 0)),
        out_shape=jax.ShapeDtypeStruct((ns, n, nh2), jnp.float32),
    )(*xs, w, b)


_CBI = 1024
_CNB = NP // _CBI


def _contrast_body(a_ref, b_ref, d_ref, lr_ref, lc_ref, rowacc, colacc):
    i = pl.program_id(0)
    j = pl.program_id(1)
    a = a_ref[...]
    b = b_ref[...]
    a = a * lax.rsqrt(jnp.maximum(jnp.sum(a * a, 1, keepdims=True), 1e-24))
    b = b * lax.rsqrt(jnp.maximum(jnp.sum(b * b, 1, keepdims=True), 1e-24))
    s = lax.dot_general(a, b, (((1,), (1,)), ((), ())),
                        preferred_element_type=jnp.float32) * (1.0 / 0.7)
    rid = i * _CBI + lax.broadcasted_iota(jnp.int32, (_CBI, _CBI), 0)
    cid = j * _CBI + lax.broadcasted_iota(jnp.int32, (_CBI, _CBI), 1)
    ex = jnp.where(cid < N, jnp.exp(s), 0.0)
    rs_ = jnp.sum(ex, axis=1, keepdims=True)
    cs_ = jnp.sum(ex * jnp.where(rid < N, 1.0, 0.0), axis=0, keepdims=True)
    zz = jnp.zeros((1, 1), jnp.float32)

    @pl.when((i == 0) & (j == 0))
    def _():
        d_ref[...] = zz
        lr_ref[...] = zz
        lc_ref[...] = zz

    @pl.when(j == 0)
    def _():
        rowacc[...] = rs_

    @pl.when(j > 0)
    def _():
        rowacc[...] += rs_

    @pl.when(i == 0)
    def _():
        colacc[j] = cs_

    @pl.when(i > 0)
    def _():
        colacc[j] += cs_

    @pl.when(i == j)
    def _():
        d_ref[...] += jnp.sum(
            jnp.where((rid == cid) & (rid < N), s, 0.0)).reshape(1, 1)

    @pl.when(j == _CNB - 1)
    def _():
        rvid = i * _CBI + lax.broadcasted_iota(jnp.int32, (_CBI, 1), 0)
        lr_ref[...] += jnp.sum(
            jnp.where(rvid < N, jnp.log(rowacc[...]), 0.0)).reshape(1, 1)

    @pl.when(i == _CNB - 1)
    def _():
        cvid = j * _CBI + lax.broadcasted_iota(jnp.int32, (1, _CBI), 1)
        lc_ref[...] += jnp.sum(
            jnp.where(cvid < N, jnp.log(colacc[j]), 0.0)).reshape(1, 1)


def _contrast_tc(a, b):
    sd = jax.ShapeDtypeStruct((1, 1), jnp.float32)
    d, lr, lc = pl.pallas_call(
        _contrast_body,
        grid=(_CNB, _CNB),
        in_specs=[
            pl.BlockSpec((_CBI, 128), lambda i, j: (i, 0)),
            pl.BlockSpec((_CBI, 128), lambda i, j: (j, 0)),
        ],
        out_specs=[pl.BlockSpec((1, 1), lambda i, j: (0, 0))] * 3,
        out_shape=[sd, sd, sd],
        scratch_shapes=[
            pltpu.VMEM((_CBI, 1), jnp.float32),
            pltpu.VMEM((_CNB, 1, _CBI), jnp.float32),
        ],
    )(a, b)
    return -d[0, 0] / N + (lr[0, 0] + lc[0, 0]) / (2.0 * N)


def _rowatt_body(c0_ref, c1_ref, x_ref, p_ref, o_ref):
    p3 = c0_ref[...]
    p2 = c1_ref[...]
    avg0 = jnp.mean(p3, axis=1, keepdims=True)
    avg1 = jnp.mean(p2, axis=1, keepdims=True)
    aa = jnp.maximum(avg0 * p_ref[0, 0] + avg1 * p_ref[0, 1] + p_ref[0, 2], 0.0)
    at0 = jax.nn.sigmoid(aa * p_ref[0, 3] + p_ref[0, 5])
    at1 = jax.nn.sigmoid(aa * p_ref[0, 4] + p_ref[0, 6])
    o_ref[...] = (p3 * (at0 * p_ref[0, 7]) + p2 * (at1 * p_ref[0, 8])
                  + p_ref[0, 9] + x_ref[...])


def _rowatt(c0, c1, x, prm, br=2048):
    return pl.pallas_call(
        _rowatt_body,
        grid=(NP // br,),
        in_specs=[
            pl.BlockSpec((br, 128), lambda i: (i, 0)),
            pl.BlockSpec((br, 128), lambda i: (i, 0)),
            pl.BlockSpec((br, 128), lambda i: (i, 0)),
            pl.BlockSpec((1, 128), lambda i: (0, 0)),
        ],
        out_specs=pl.BlockSpec((br, 128), lambda i: (i, 0)),
        out_shape=jax.ShapeDtypeStruct((NP, 128), jnp.float32),
    )(c0, c1, x, prm)


def _mlp_body(g_ref, w1_ref, b1_ref, w2_ref, b2_ref, w3_ref, b3_ref, o_ref):
    h = (jnp.dot(g_ref[0], w1_ref[0], preferred_element_type=jnp.float32)
         + jnp.dot(g_ref[1], w1_ref[1], preferred_element_type=jnp.float32)
         + b1_ref[...])
    h = jnp.where(h > 0, h, 0.01 * h)
    h = jnp.dot(h, w2_ref[...], preferred_element_type=jnp.float32) + b2_ref[...]
    h = jnp.where(h > 0, h, 0.01 * h)
    o_ref[...] = (jnp.dot(h, w3_ref[...], preferred_element_type=jnp.float32)
                  + b3_ref[...])


def _mlp(g, w1, b1, w2, b2, w3, b3, br=1024):
    nb = 4096 // br
    return pl.pallas_call(
        _mlp_body,
        grid=(nb,),
        in_specs=[
            pl.BlockSpec((2, br, 128), lambda i: (0, i, 0)),
            pl.BlockSpec((2, 128, 64), lambda i: (0, 0, 0)),
            pl.BlockSpec((1, 64), lambda i: (0, 0)),
            pl.BlockSpec((64, 32), lambda i: (0, 0)),
            pl.BlockSpec((1, 32), lambda i: (0, 0)),
            pl.BlockSpec((32, 1), lambda i: (0, 0)),
            pl.BlockSpec((1, 1), lambda i: (0, 0)),
        ],
        out_specs=pl.BlockSpec((br, 1), lambda i: (i, 0)),
        out_shape=jax.ShapeDtypeStruct((4096, 1), jnp.float32),
    )(g, w1, b1, w2, b2, w3, b3)


def _tiled(x):
    return x.reshape(NTILES, NCHUNK, CHUNK)


def _he_plan(he):
    ni, ei = he[0], he[1]
    return {
        "nhalf": ni.reshape(NCORES, NTILES, ECH, CHUNK),
        "ehalf": ei.reshape(NCORES, NTILES, ECH, CHUNK),
    }


_BN_S = 1.0 / np.sqrt(1.0 + BN_EPS)


def _conv_slab(table, plan, binv_col, dinv_col, bias_row, zb128):
    pa = _sc_seghalf(table, plan["nhalf"], plan["ehalf"])
    ef = _merge(pa.reshape(2, NP, 128), binv_col, zb128)
    pb = _sc_seghalf(ef, plan["ehalf"], plan["nhalf"])
    return _merge(pb.reshape(2, NP, 128), dinv_col, bias_row)


def _run_hgnn(x_pad, plan, pp, binv_col, dinv_col):
    zb128 = jnp.zeros((1, 128), jnp.float32)
    w1 = pp["W1"].reshape(1, 128, 2, 128)
    b0 = jnp.zeros((2, 1, 128), jnp.float32)
    xl1 = _dense([x_pad], w1, b0)
    h1 = [
        _conv_slab(xl1[h], plan, binv_col, dinv_col,
                   pp["b1"][h * 128:(h + 1) * 128].reshape(1, 128), zb128)
        for h in range(2)
    ]
    s1 = pp["g1"] * _BN_S
    w2p = (s1[:, None] * pp["W2"]).reshape(2, 128, 1, 128)
    b2p = (pp["be1"] @ pp["W2"]).reshape(1, 1, 128)
    xl2 = _dense(h1, w2p, b2p)
    return _conv_slab(xl2[0], plan, binv_col, dinv_col,
                      pp["b2"].reshape(1, 128), zb128)


def _att_prm(ap):
    v = jnp.stack([ap["fc1W"][0, 0], ap["fc1W"][1, 0], ap["fc1b"][0],
                   ap["fc2W"][0, 0], ap["fc2W"][0, 1],
                   ap["fc2b"][0], ap["fc2b"][1],
                   ap["convW"][0], ap["convW"][1], ap["convb"][0]])
    return jnp.zeros((1, 128), jnp.float32).at[0, :10].set(v)


def kernel(x_protein, hyperedge_protein_index, x_meta, hyperedge_meta_index,
           index, params):
    he_p, he_m = hyperedge_protein_index, hyperedge_meta_index
    idx8 = jnp.stack([
        _tiled(he_p[0, 0]), _tiled(he_p[0, 1]),
        _tiled(he_p[1, 0]), _tiled(he_p[1, 1]),
        _tiled(he_m[0, 0]), _tiled(he_m[0, 1]),
        _tiled(he_m[1, 0]), _tiled(he_m[1, 1]),
    ], axis=0)
    inv = _sc_degrees(idx8)

    xp_pad = jnp.pad(x_protein, ((0, NP - N), (0, 0)))
    xm_pad = jnp.pad(x_meta, ((0, NP - N), (0, 0)))

    plans = [_he_plan(he_p[0]), _he_plan(he_p[1]),
             _he_plan(he_m[0]), _he_plan(he_m[1])]

    def col(j):
        return inv[j].reshape(NP, 1)

    p2 = _run_hgnn(xp_pad, plans[0], params["pc1"], col(1), col(0))
    p3 = _run_hgnn(xp_pad, plans[1], params["pc2"], col(3), col(2))
    m2 = _run_hgnn(xm_pad, plans[2], params["mc1"], col(5), col(4))
    m3 = _run_hgnn(xm_pad, plans[3], params["mc2"], col(7), col(6))

    loss_p = _contrast_tc(p2, p3)
    loss_m = _contrast_tc(m2, m3)
    loss = (jnp.exp(-params["p12"]) * loss_p + params["p12"]
            + jnp.exp(-params["m13"]) * loss_m + params["m13"])

    protein = _rowatt(p3, p2, xp_pad, _att_prm(params["attp"]))
    meta = _rowatt(m3, m2, xm_pad, _att_prm(params["attm"]))

    tables = jnp.concatenate([protein, meta], axis=0)
    idx_all = jnp.concatenate([index[0], index[1] + NP]).reshape(32, 2, 128)
    g = _sc_gather(tables, idx_all)

    e = params["enc"]
    s1 = e["g1"] * _BN_S
    w1 = (e["W1"] * s1[None, :]).reshape(2, 128, 64)
    b1 = (e["b1"] * s1 + e["be1"]).reshape(1, 64)
    s2 = e["g2"] * _BN_S
    w2 = e["W2"] * s2[None, :]
    b2 = (e["b2"] * s2 + e["be2"]).reshape(1, 32)
    out = _mlp(g.reshape(2, 4096, 128), w1, b1, w2, b2,
               e["W3"], e["b3"].reshape(1, 1))
    return out, loss


# speedup vs baseline: 5.1670x; 1.0047x over previous
import functools

import jax
import jax.numpy as jnp
import numpy as np
from jax import lax
from jax.experimental import pallas as pl
from jax.experimental.pallas import tpu as pltpu
from jax.experimental.pallas import tpu_sc as plsc

N = 10000
NP = 10240
E = 160000
BN_EPS = 1e-5
NTILES = 16
NCORES = 2
CHUNK = 125
NCHUNK = E // NTILES // CHUNK
SCHUNK = 100
SCH = E // NCORES // NTILES // SCHUNK
SSUP = 25
RPT = NP // NTILES
DH2 = 128


@functools.lru_cache(maxsize=None)
def _mesh():
    return plsc.VectorSubcoreMesh(core_axis_name="c", subcore_axis_name="s")


@functools.lru_cache(maxsize=None)
def _make_degrees():
    return functools.partial(
        pl.kernel,
        mesh=_mesh(),
        out_type=jax.ShapeDtypeStruct((8, NP), jnp.float32),
        scratch_types=[
            pltpu.VMEM((NCHUNK, CHUNK), jnp.int32),
            pltpu.VMEM((CHUNK,), jnp.float32),
            pltpu.VMEM((RPT,), jnp.float32),
            pltpu.VMEM_SHARED((NP,), jnp.float32),
            pltpu.SemaphoreType.DMA,
        ],
    )(_sc_degrees_body)


def _sc_degrees_body(idx_hbm, ones_hbm, zeros_hbm, inv_hbm,
                     idx_v, ones_v, val_v, acc, sem):
    c = lax.axis_index("c")
    s = lax.axis_index("s")
    pltpu.sync_copy(ones_hbm, ones_v)
    for jb in range(4):
        job = c * 4 + jb
        pltpu.sync_copy(zeros_hbm, acc.at[pl.ds(s * RPT, RPT)])
        pltpu.sync_copy(idx_hbm.at[job, s], idx_v)
        plsc.subcore_barrier()

        def grp(g, _):
            descs = []
            for u in range(8):
                descs.append(pltpu.async_copy(
                    ones_v, acc.at[idx_v.at[g * 8 + u]], sem, add=True))
            for d in descs:
                d.wait()
            return ()

        lax.fori_loop(0, NCHUNK // 8, grp, ())
        plsc.subcore_barrier()
        pltpu.sync_copy(acc.at[pl.ds(s * RPT, RPT)], val_v)
        for l in range(RPT // 16):
            sl = pl.ds(l * 16, 16)
            v = val_v[sl]
            val_v[sl] = jnp.where(v > 0.0, 1.0 / v, 0.0)
        pltpu.sync_copy(val_v, inv_hbm.at[job, pl.ds(s * RPT, RPT)])
        plsc.subcore_barrier()


def _sc_degrees(idx8):
    ones = jnp.ones((CHUNK,), jnp.float32)
    zeros = jnp.zeros((RPT,), jnp.float32)
    return _make_degrees()(idx8, ones, zeros)


@functools.lru_cache(maxsize=None)
def _make_seghalf():
    @functools.partial(
        pl.kernel,
        mesh=_mesh(),
        out_type=jax.ShapeDtypeStruct((NCORES * NP, DH2), jnp.float32),
        scratch_types=[
            pltpu.VMEM((SSUP, SCHUNK), jnp.int32),
            pltpu.VMEM((SSUP, SCHUNK), jnp.int32),
            pltpu.VMEM((SCHUNK, DH2), jnp.float32),
            pltpu.VMEM((SCHUNK, DH2), jnp.float32),
            pltpu.VMEM((SCHUNK, DH2), jnp.float32),
            pltpu.VMEM_SHARED((NP, DH2), jnp.float32),
            pltpu.SemaphoreType.DMA,
            pltpu.SemaphoreType.DMA,
        ],
    )
    def seghalf(table, src, dst, zeros, out,
                src_v, dst_v, buf0, buf1, buf2, acc, gsem, ssem):
        c = lax.axis_index("c")
        s = lax.axis_index("s")
        bufs = (buf0, buf1, buf2)

        def gwait(j, b):
            pltpu.make_async_copy(
                table.at[src_v.at[j]], bufs[b], gsem).wait()

        def swait(j, b):
            pltpu.make_async_copy(
                bufs[b], acc.at[dst_v.at[j]], ssem).wait()

        pltpu.sync_copy(zeros, acc.at[pl.ds(s * RPT, RPT)])
        plsc.subcore_barrier()

        for si in range(SCH // SSUP):
            pltpu.sync_copy(src.at[c, s, si], src_v)
            pltpu.sync_copy(dst.at[c, s, si], dst_v)
            pltpu.async_copy(table.at[src_v.at[0]], buf0, gsem)
            pltpu.async_copy(table.at[src_v.at[1]], buf1, gsem)

            def step(j3, _):
                for u in range(3):
                    l = j3 * 3 + u
                    gwait(l, u)
                    pltpu.async_copy(bufs[u], acc.at[dst_v.at[l]], ssem,
                                     add=True)

                    @pl.when(l >= 1)
                    def _():
                        swait(l - 1, (u + 2) % 3)

                    @pl.when(l + 2 < SSUP)
                    def _():
                        pltpu.async_copy(
                            table.at[src_v.at[l + 2]], bufs[(u + 2) % 3],
                            gsem)
                return ()

            lax.fori_loop(0, (SSUP - 1) // 3, step, ())
            l = SSUP - 1
            gwait(l, l % 3)
            pltpu.async_copy(bufs[l % 3], acc.at[dst_v.at[l]], ssem, add=True)
            swait(l - 1, (l - 1) % 3)
            swait(l, l % 3)
        plsc.subcore_barrier()
        pltpu.sync_copy(acc.at[pl.ds(s * RPT, RPT)],
                        out.at[pl.ds(c * NP + s * RPT, RPT)])

    return seghalf


def _sc_seghalf(table, src, dst):
    zeros = jnp.zeros((RPT, DH2), jnp.float32)
    return _make_seghalf()(table, src, dst, zeros)


@functools.lru_cache(maxsize=None)
def _make_gather():
    return functools.partial(
        pl.kernel,
        mesh=_mesh(),
        out_type=jax.ShapeDtypeStruct((8192, 128), jnp.float32),
        scratch_types=[
            pltpu.VMEM((2, 128), jnp.int32),
            pltpu.VMEM((128, 128), jnp.float32),
            pltpu.VMEM((128, 128), jnp.float32),
            pltpu.SemaphoreType.DMA,
        ],
    )(_sc_gather_body)


def _sc_gather(tables, idx_all):
    return _make_gather()(tables, idx_all)


def _sc_gather_body(tab, idx, out, idx_v, bufa, bufb, sem):
    c = lax.axis_index("c")
    s = lax.axis_index("s")
    w = s * NCORES + c
    pltpu.sync_copy(idx.at[w], idx_v)
    da = pltpu.async_copy(tab.at[idx_v.at[0]], bufa, sem)
    db = pltpu.async_copy(tab.at[idx_v.at[1]], bufb, sem)
    da.wait()
    pltpu.sync_copy(bufa, out.at[pl.ds(w * 256, 128)])
    db.wait()
    pltpu.sync_copy(bufb, out.at[pl.ds(w * 256 + 128, 128)])


def _merge_body(p_ref, rs_ref, b_ref, o_ref):
    o_ref[...] = (p_ref[0] + p_ref[1]) * rs_ref[...] + b_ref[...]


def _merge(parts, rowscale, bias, br=2048):
    return pl.pallas_call(
        _merge_body,
        grid=(NP // br,),
        in_specs=[
            pl.BlockSpec((2, br, 128), lambda i: (0, i, 0)),
            pl.BlockSpec((br, 1), lambda i: (i, 0)),
            pl.BlockSpec((1, 128), lambda i: (0, 0)),
        ],
        out_specs=pl.BlockSpec((br, 128), lambda i: (i, 0)),
        out_shape=jax.ShapeDtypeStruct((NP, 128), jnp.float32),
    )(parts, rowscale, bias)


def _dense(xs, w, b, br=2048):
    kc = len(xs)
    n, k2 = xs[0].shape
    ns, nh2 = w.shape[2], w.shape[3]

    def body(*refs):
        x_refs = refs[:kc]
        w_ref, b_ref, o_ref = refs[kc], refs[kc + 1], refs[kc + 2]
        for h in range(ns):
            acc = jnp.dot(x_refs[0][...], w_ref[0, :, h],
                          preferred_element_type=jnp.float32)
            for k in range(1, kc):
                acc += jnp.dot(x_refs[k][...], w_ref[k, :, h],
                               preferred_element_type=jnp.float32)
            o_ref[h] = acc + b_ref[h]

    return pl.pallas_call(
        body,
        grid=(n // br,),
        in_specs=[pl.BlockSpec((br, k2), lambda i: (i, 0))] * kc + [
            pl.BlockSpec((kc, k2, ns, nh2), lambda i: (0, 0, 0, 0)),
            pl.BlockSpec((ns, 1, nh2), lambda i: (0, 0, 0)),
        ],
        out_specs=pl.BlockSpec((ns, br, nh2), lambda i: (0, i, 0)),
        out_shape=jax.ShapeDtypeStruct((ns, n, nh2), jnp.float32),
    )(*xs, w, b)


_CBI = 1024
_CNB = NP // _CBI


def _contrast_body(a_ref, b_ref, d_ref, lr_ref, lc_ref, rowacc, colacc):
    i = pl.program_id(0)
    j = pl.program_id(1)
    a = a_ref[...]
    b = b_ref[...]
    a = a * lax.rsqrt(jnp.maximum(jnp.sum(a * a, 1, keepdims=True), 1e-24))
    b = b * lax.rsqrt(jnp.maximum(jnp.sum(b * b, 1, keepdims=True), 1e-24))
    s = lax.dot_general(a.astype(jnp.bfloat16), b.astype(jnp.bfloat16),
                        (((1,), (1,)), ((), ())),
                        preferred_element_type=jnp.float32) * (1.0 / 0.7)
    rid = i * _CBI + lax.broadcasted_iota(jnp.int32, (_CBI, _CBI), 0)
    cid = j * _CBI + lax.broadcasted_iota(jnp.int32, (_CBI, _CBI), 1)
    ex = jnp.where(cid < N, jnp.exp(s), 0.0)
    rs_ = jnp.sum(ex, axis=1, keepdims=True)
    cs_ = jnp.sum(ex * jnp.where(rid < N, 1.0, 0.0), axis=0, keepdims=True)
    zz = jnp.zeros((1, 1), jnp.float32)

    @pl.when((i == 0) & (j == 0))
    def _():
        d_ref[...] = zz
        lr_ref[...] = zz
        lc_ref[...] = zz

    @pl.when(j == 0)
    def _():
        rowacc[...] = rs_

    @pl.when(j > 0)
    def _():
        rowacc[...] += rs_

    @pl.when(i == 0)
    def _():
        colacc[j] = cs_

    @pl.when(i > 0)
    def _():
        colacc[j] += cs_

    @pl.when(i == j)
    def _():
        d_ref[...] += jnp.sum(
            jnp.where((rid == cid) & (rid < N), s, 0.0)).reshape(1, 1)

    @pl.when(j == _CNB - 1)
    def _():
        rvid = i * _CBI + lax.broadcasted_iota(jnp.int32, (_CBI, 1), 0)
        lr_ref[...] += jnp.sum(
            jnp.where(rvid < N, jnp.log(rowacc[...]), 0.0)).reshape(1, 1)

    @pl.when(i == _CNB - 1)
    def _():
        cvid = j * _CBI + lax.broadcasted_iota(jnp.int32, (1, _CBI), 1)
        lc_ref[...] += jnp.sum(
            jnp.where(cvid < N, jnp.log(colacc[j]), 0.0)).reshape(1, 1)


def _contrast_tc(a, b):
    sd = jax.ShapeDtypeStruct((1, 1), jnp.float32)
    d, lr, lc = pl.pallas_call(
        _contrast_body,
        grid=(_CNB, _CNB),
        in_specs=[
            pl.BlockSpec((_CBI, 128), lambda i, j: (i, 0)),
            pl.BlockSpec((_CBI, 128), lambda i, j: (j, 0)),
        ],
        out_specs=[pl.BlockSpec((1, 1), lambda i, j: (0, 0))] * 3,
        out_shape=[sd, sd, sd],
        scratch_shapes=[
            pltpu.VMEM((_CBI, 1), jnp.float32),
            pltpu.VMEM((_CNB, 1, _CBI), jnp.float32),
        ],
    )(a, b)
    return -d[0, 0] / N + (lr[0, 0] + lc[0, 0]) / (2.0 * N)


def _rowatt_body(c0_ref, c1_ref, x_ref, p_ref, o_ref):
    p3 = c0_ref[...]
    p2 = c1_ref[...]
    avg0 = jnp.mean(p3, axis=1, keepdims=True)
    avg1 = jnp.mean(p2, axis=1, keepdims=True)
    aa = jnp.maximum(avg0 * p_ref[0, 0] + avg1 * p_ref[0, 1] + p_ref[0, 2], 0.0)
    at0 = jax.nn.sigmoid(aa * p_ref[0, 3] + p_ref[0, 5])
    at1 = jax.nn.sigmoid(aa * p_ref[0, 4] + p_ref[0, 6])
    o_ref[...] = (p3 * (at0 * p_ref[0, 7]) + p2 * (at1 * p_ref[0, 8])
                  + p_ref[0, 9] + x_ref[...])


def _rowatt(c0, c1, x, prm, br=2048):
    return pl.pallas_call(
        _rowatt_body,
        grid=(NP // br,),
        in_specs=[
            pl.BlockSpec((br, 128), lambda i: (i, 0)),
            pl.BlockSpec((br, 128), lambda i: (i, 0)),
            pl.BlockSpec((br, 128), lambda i: (i, 0)),
            pl.BlockSpec((1, 128), lambda i: (0, 0)),
        ],
        out_specs=pl.BlockSpec((br, 128), lambda i: (i, 0)),
        out_shape=jax.ShapeDtypeStruct((NP, 128), jnp.float32),
    )(c0, c1, x, prm)


def _mlp_body(g_ref, w1_ref, b1_ref, w2_ref, b2_ref, w3_ref, b3_ref, o_ref):
    h = (jnp.dot(g_ref[0], w1_ref[0], preferred_element_type=jnp.float32)
         + jnp.dot(g_ref[1], w1_ref[1], preferred_element_type=jnp.float32)
         + b1_ref[...])
    h = jnp.where(h > 0, h, 0.01 * h)
    h = jnp.dot(h, w2_ref[...], preferred_element_type=jnp.float32) + b2_ref[...]
    h = jnp.where(h > 0, h, 0.01 * h)
    o_ref[...] = (jnp.dot(h, w3_ref[...], preferred_element_type=jnp.float32)
                  + b3_ref[...])


def _mlp(g, w1, b1, w2, b2, w3, b3, br=1024):
    nb = 4096 // br
    return pl.pallas_call(
        _mlp_body,
        grid=(nb,),
        in_specs=[
            pl.BlockSpec((2, br, 128), lambda i: (0, i, 0)),
            pl.BlockSpec((2, 128, 64), lambda i: (0, 0, 0)),
            pl.BlockSpec((1, 64), lambda i: (0, 0)),
            pl.BlockSpec((64, 32), lambda i: (0, 0)),
            pl.BlockSpec((1, 32), lambda i: (0, 0)),
            pl.BlockSpec((32, 1), lambda i: (0, 0)),
            pl.BlockSpec((1, 1), lambda i: (0, 0)),
        ],
        out_specs=pl.BlockSpec((br, 1), lambda i: (i, 0)),
        out_shape=jax.ShapeDtypeStruct((4096, 1), jnp.float32),
    )(g, w1, b1, w2, b2, w3, b3)


def _tiled(x):
    return x.reshape(NTILES, NCHUNK, CHUNK)


def _he_plan(he):
    ni, ei = he[0], he[1]
    return {
        "nhalf": ni.reshape(NCORES, NTILES, SCH // SSUP, SSUP, SCHUNK),
        "ehalf": ei.reshape(NCORES, NTILES, SCH // SSUP, SSUP, SCHUNK),
    }


_BN_S = 1.0 / np.sqrt(1.0 + BN_EPS)


def _conv_slab(table, plan, binv_col, dinv_col, bias_row, zb128):
    pa = _sc_seghalf(table, plan["nhalf"], plan["ehalf"])
    ef = _merge(pa.reshape(2, NP, 128), binv_col, zb128)
    pb = _sc_seghalf(ef, plan["ehalf"], plan["nhalf"])
    return _merge(pb.reshape(2, NP, 128), dinv_col, bias_row)


def _run_hgnn(x_pad, plan, pp, binv_col, dinv_col):
    zb128 = jnp.zeros((1, 128), jnp.float32)
    w1 = pp["W1"].reshape(1, 128, 2, 128)
    b0 = jnp.zeros((2, 1, 128), jnp.float32)
    xl1 = _dense([x_pad], w1, b0)
    h1 = [
        _conv_slab(xl1[h], plan, binv_col, dinv_col,
                   pp["b1"][h * 128:(h + 1) * 128].reshape(1, 128), zb128)
        for h in range(2)
    ]
    s1 = pp["g1"] * _BN_S
    w2p = (s1[:, None] * pp["W2"]).reshape(2, 128, 1, 128)
    b2p = (pp["be1"] @ pp["W2"]).reshape(1, 1, 128)
    xl2 = _dense(h1, w2p, b2p)
    return _conv_slab(xl2[0], plan, binv_col, dinv_col,
                      pp["b2"].reshape(1, 128), zb128)


def _att_prm(ap):
    v = jnp.stack([ap["fc1W"][0, 0], ap["fc1W"][1, 0], ap["fc1b"][0],
                   ap["fc2W"][0, 0], ap["fc2W"][0, 1],
                   ap["fc2b"][0], ap["fc2b"][1],
                   ap["convW"][0], ap["convW"][1], ap["convb"][0]])
    return jnp.zeros((1, 128), jnp.float32).at[0, :10].set(v)


def kernel(x_protein, hyperedge_protein_index, x_meta, hyperedge_meta_index,
           index, params):
    he_p, he_m = hyperedge_protein_index, hyperedge_meta_index
    idx8 = jnp.stack([
        _tiled(he_p[0, 0]), _tiled(he_p[0, 1]),
        _tiled(he_p[1, 0]), _tiled(he_p[1, 1]),
        _tiled(he_m[0, 0]), _tiled(he_m[0, 1]),
        _tiled(he_m[1, 0]), _tiled(he_m[1, 1]),
    ], axis=0)
    inv = _sc_degrees(idx8)

    xp_pad = jnp.pad(x_protein, ((0, NP - N), (0, 0)))
    xm_pad = jnp.pad(x_meta, ((0, NP - N), (0, 0)))

    plans = [_he_plan(he_p[0]), _he_plan(he_p[1]),
             _he_plan(he_m[0]), _he_plan(he_m[1])]

    def col(j):
        return inv[j].reshape(NP, 1)

    p2 = _run_hgnn(xp_pad, plans[0], params["pc1"], col(1), col(0))
    p3 = _run_hgnn(xp_pad, plans[1], params["pc2"], col(3), col(2))
    m2 = _run_hgnn(xm_pad, plans[2], params["mc1"], col(5), col(4))
    m3 = _run_hgnn(xm_pad, plans[3], params["mc2"], col(7), col(6))

    loss_p = _contrast_tc(p2, p3)
    loss_m = _contrast_tc(m2, m3)
    loss = (jnp.exp(-params["p12"]) * loss_p + params["p12"]
            + jnp.exp(-params["m13"]) * loss_m + params["m13"])

    protein = _rowatt(p3, p2, xp_pad, _att_prm(params["attp"]))
    meta = _rowatt(m3, m2, xm_pad, _att_prm(params["attm"]))

    tables = jnp.concatenate([protein, meta], axis=0)
    idx_all = jnp.concatenate([index[0], index[1] + NP]).reshape(32, 2, 128)
    g = _sc_gather(tables, idx_all)

    e = params["enc"]
    s1 = e["g1"] * _BN_S
    w1 = (e["W1"] * s1[None, :]).reshape(2, 128, 64)
    b1 = (e["b1"] * s1 + e["be1"]).reshape(1, 64)
    s2 = e["g2"] * _BN_S
    w2 = e["W2"] * s2[None, :]
    b2 = (e["b2"] * s2 + e["be2"]).reshape(1, 32)
    out = _mlp(g.reshape(2, 4096, 128), w1, b1, w2, b2,
               e["W3"], e["b3"].reshape(1, 1))
    return out, loss


# speedup vs baseline: 5.1960x; 1.0056x over previous
import functools

import jax
import jax.numpy as jnp
import numpy as np
from jax import lax
from jax.experimental import pallas as pl
from jax.experimental.pallas import tpu as pltpu
from jax.experimental.pallas import tpu_sc as plsc

N = 10000
NP = 10240
E = 160000
BN_EPS = 1e-5
NTILES = 16
NCORES = 2
CHUNK = 125
NCHUNK = E // NTILES // CHUNK
SCHUNK = 100
SCH = E // NCORES // NTILES // SCHUNK
SSUP = 25
RPT = NP // NTILES
DH2 = 128


@functools.lru_cache(maxsize=None)
def _mesh():
    return plsc.VectorSubcoreMesh(core_axis_name="c", subcore_axis_name="s")


@functools.lru_cache(maxsize=None)
def _make_degrees():
    return functools.partial(
        pl.kernel,
        mesh=_mesh(),
        out_type=jax.ShapeDtypeStruct((8, NP), jnp.float32),
        scratch_types=[
            pltpu.VMEM((NCHUNK, CHUNK), jnp.int32),
            pltpu.VMEM((CHUNK,), jnp.float32),
            pltpu.VMEM((RPT,), jnp.float32),
            pltpu.VMEM_SHARED((NP,), jnp.float32),
            pltpu.SemaphoreType.DMA,
        ],
    )(_sc_degrees_body)


def _sc_degrees_body(idx_hbm, ones_hbm, zeros_hbm, inv_hbm,
                     idx_v, ones_v, val_v, acc, sem):
    c = lax.axis_index("c")
    s = lax.axis_index("s")
    pltpu.sync_copy(ones_hbm, ones_v)
    for jb in range(4):
        job = c * 4 + jb
        pltpu.sync_copy(zeros_hbm, acc.at[pl.ds(s * RPT, RPT)])
        pltpu.sync_copy(idx_hbm.at[job, s], idx_v)
        plsc.subcore_barrier()

        def grp(g, _):
            descs = []
            for u in range(8):
                descs.append(pltpu.async_copy(
                    ones_v, acc.at[idx_v.at[g * 8 + u]], sem, add=True))
            for d in descs:
                d.wait()
            return ()

        lax.fori_loop(0, NCHUNK // 8, grp, ())
        plsc.subcore_barrier()
        pltpu.sync_copy(acc.at[pl.ds(s * RPT, RPT)], val_v)
        for l in range(RPT // 16):
            sl = pl.ds(l * 16, 16)
            v = val_v[sl]
            val_v[sl] = jnp.where(v > 0.0, 1.0 / v, 0.0)
        pltpu.sync_copy(val_v, inv_hbm.at[job, pl.ds(s * RPT, RPT)])
        plsc.subcore_barrier()


def _sc_degrees(idx8):
    ones = jnp.ones((CHUNK,), jnp.float32)
    zeros = jnp.zeros((RPT,), jnp.float32)
    return _make_degrees()(idx8, ones, zeros)


@functools.lru_cache(maxsize=None)
def _make_seghalf():
    @functools.partial(
        pl.kernel,
        mesh=_mesh(),
        out_type=jax.ShapeDtypeStruct((NCORES * NP, DH2), jnp.float32),
        scratch_types=[
            pltpu.VMEM((SSUP, SCHUNK), jnp.int32),
            pltpu.VMEM((SSUP, SCHUNK), jnp.int32),
            pltpu.VMEM((SCHUNK, DH2), jnp.float32),
            pltpu.VMEM((SCHUNK, DH2), jnp.float32),
            pltpu.VMEM((SCHUNK, DH2), jnp.float32),
            pltpu.VMEM_SHARED((NP, DH2), jnp.float32),
            pltpu.SemaphoreType.DMA,
            pltpu.SemaphoreType.DMA,
        ],
    )
    def seghalf(table, src, dst, zeros, out,
                src_v, dst_v, buf0, buf1, buf2, acc, gsem, ssem):
        c = lax.axis_index("c")
        s = lax.axis_index("s")
        bufs = (buf0, buf1, buf2)

        def gwait(j, b):
            pltpu.make_async_copy(
                table.at[src_v.at[j]], bufs[b], gsem).wait()

        def swait(j, b):
            pltpu.make_async_copy(
                bufs[b], acc.at[dst_v.at[j]], ssem).wait()

        pltpu.sync_copy(zeros, acc.at[pl.ds(s * RPT, RPT)])
        plsc.subcore_barrier()

        for si in range(SCH // SSUP):
            pltpu.sync_copy(src.at[c, s, si], src_v)
            pltpu.sync_copy(dst.at[c, s, si], dst_v)
            pltpu.async_copy(table.at[src_v.at[0]], buf0, gsem)
            pltpu.async_copy(table.at[src_v.at[1]], buf1, gsem)

            def step(j3, _):
                for u in range(3):
                    l = j3 * 3 + u
                    gwait(l, u)
                    pltpu.async_copy(bufs[u], acc.at[dst_v.at[l]], ssem,
                                     add=True)

                    @pl.when(l >= 1)
                    def _():
                        swait(l - 1, (u + 2) % 3)

                    @pl.when(l + 2 < SSUP)
                    def _():
                        pltpu.async_copy(
                            table.at[src_v.at[l + 2]], bufs[(u + 2) % 3],
                            gsem)
                return ()

            lax.fori_loop(0, (SSUP - 1) // 3, step, ())
            l = SSUP - 1
            gwait(l, l % 3)
            pltpu.async_copy(bufs[l % 3], acc.at[dst_v.at[l]], ssem, add=True)
            swait(l - 1, (l - 1) % 3)
            swait(l, l % 3)
        plsc.subcore_barrier()
        pltpu.sync_copy(acc.at[pl.ds(s * RPT, RPT)],
                        out.at[pl.ds(c * NP + s * RPT, RPT)])

    return seghalf


def _sc_seghalf(table, src, dst):
    zeros = jnp.zeros((RPT, DH2), jnp.float32)
    return _make_seghalf()(table, src, dst, zeros)


@functools.lru_cache(maxsize=None)
def _make_gather():
    return functools.partial(
        pl.kernel,
        mesh=_mesh(),
        out_type=jax.ShapeDtypeStruct((8192, 128), jnp.float32),
        scratch_types=[
            pltpu.VMEM((2, 128), jnp.int32),
            pltpu.VMEM((128, 128), jnp.float32),
            pltpu.VMEM((128, 128), jnp.float32),
            pltpu.SemaphoreType.DMA,
        ],
    )(_sc_gather_body)


def _sc_gather(tables, idx_all):
    return _make_gather()(tables, idx_all)


def _sc_gather_body(tab, idx, out, idx_v, bufa, bufb, sem):
    c = lax.axis_index("c")
    s = lax.axis_index("s")
    w = s * NCORES + c
    pltpu.sync_copy(idx.at[w], idx_v)
    da = pltpu.async_copy(tab.at[idx_v.at[0]], bufa, sem)
    db = pltpu.async_copy(tab.at[idx_v.at[1]], bufb, sem)
    da.wait()
    pltpu.sync_copy(bufa, out.at[pl.ds(w * 256, 128)])
    db.wait()
    pltpu.sync_copy(bufb, out.at[pl.ds(w * 256 + 128, 128)])


def _merge_body(p_ref, rs_ref, b_ref, o_ref):
    o_ref[...] = (p_ref[0] + p_ref[1]) * rs_ref[...] + b_ref[...]


def _merge(parts, rowscale, bias, br=2048):
    return pl.pallas_call(
        _merge_body,
        grid=(NP // br,),
        in_specs=[
            pl.BlockSpec((2, br, 128), lambda i: (0, i, 0)),
            pl.BlockSpec((br, 1), lambda i: (i, 0)),
            pl.BlockSpec((1, 128), lambda i: (0, 0)),
        ],
        out_specs=pl.BlockSpec((br, 128), lambda i: (i, 0)),
        out_shape=jax.ShapeDtypeStruct((NP, 128), jnp.float32),
    )(parts, rowscale, bias)


def _dense(xs, w, b, br=2048):
    kc = len(xs)
    n, k2 = xs[0].shape
    ns, nh2 = w.shape[2], w.shape[3]

    def body(*refs):
        x_refs = refs[:kc]
        w_ref, b_ref, o_ref = refs[kc], refs[kc + 1], refs[kc + 2]
        for h in range(ns):
            acc = jnp.dot(x_refs[0][...], w_ref[0, :, h],
                          preferred_element_type=jnp.float32)
            for k in range(1, kc):
                acc += jnp.dot(x_refs[k][...], w_ref[k, :, h],
                               preferred_element_type=jnp.float32)
            o_ref[h] = acc + b_ref[h]

    return pl.pallas_call(
        body,
        grid=(n // br,),
        in_specs=[pl.BlockSpec((br, k2), lambda i: (i, 0))] * kc + [
            pl.BlockSpec((kc, k2, ns, nh2), lambda i: (0, 0, 0, 0)),
            pl.BlockSpec((ns, 1, nh2), lambda i: (0, 0, 0)),
        ],
        out_specs=pl.BlockSpec((ns, br, nh2), lambda i: (0, i, 0)),
        out_shape=jax.ShapeDtypeStruct((ns, n, nh2), jnp.float32),
    )(*xs, w, b)


_CBI = 1024
_CNB = NP // _CBI


def _make_band_body(iband):
    def body(a_ref, b_ref, rm_ref, cm_ref, rmc_ref, cs_ref, d_ref, lr_ref,
             rowacc):
        j = pl.program_id(0)
        a = a_ref[...]
        b = b_ref[...]
        a = a * lax.rsqrt(jnp.maximum(jnp.sum(a * a, 1, keepdims=True),
                                      1e-24))
        b = b * lax.rsqrt(jnp.maximum(jnp.sum(b * b, 1, keepdims=True),
                                      1e-24))
        s = lax.dot_general(a.astype(jnp.bfloat16), b.astype(jnp.bfloat16),
                            (((1,), (1,)), ((), ())),
                            preferred_element_type=jnp.float32) * (1.0 / 0.7)
        ex = jnp.exp(s) * cm_ref[...]
        rs_ = jnp.sum(ex, axis=1, keepdims=True)
        cs_ref[...] = jnp.sum(ex * rm_ref[...], axis=0, keepdims=True)

        @pl.when(j == 0)
        def _():
            rowacc[...] = rs_
            d_ref[...] = jnp.zeros((1, 1), jnp.float32)

        @pl.when(j > 0)
        def _():
            rowacc[...] += rs_

        @pl.when(j == iband)
        def _():
            rid = lax.broadcasted_iota(jnp.int32, (_CBI, _CBI), 0)
            cid = lax.broadcasted_iota(jnp.int32, (_CBI, _CBI), 1)
            gid = iband * _CBI + rid
            d_ref[...] += jnp.sum(
                jnp.where((rid == cid) & (gid < N), s, 0.0)).reshape(1, 1)

        @pl.when(j == _CNB - 1)
        def _():
            lr_ref[...] = jnp.sum(
                rmc_ref[...] * jnp.log(rowacc[...])).reshape(1, 1)

    return body


def _colsum_fin_body(cs_ref, m_ref, o_ref):
    tot = cs_ref[0]
    for k in range(1, _CNB):
        tot = tot + cs_ref[k]
    j = pl.program_id(0)
    part = jnp.sum(jnp.where(m_ref[...] > 0.0, jnp.log(tot),
                             0.0)).reshape(1, 1)

    @pl.when(j == 0)
    def _():
        o_ref[...] = part

    @pl.when(j > 0)
    def _():
        o_ref[...] += part


def _contrast_tc(a, b, vmask):
    sd = jax.ShapeDtypeStruct((1, 1), jnp.float32)
    vmask_row = vmask.reshape(1, NP)
    cs_bands, d_parts, lr_parts = [], [], []
    for iband in range(_CNB):
        cs, d, lr = pl.pallas_call(
            _make_band_body(iband),
            grid=(_CNB,),
            in_specs=[
                pl.BlockSpec((_CBI, 128), lambda j, _i=iband: (_i, 0)),
                pl.BlockSpec((_CBI, 128), lambda j: (j, 0)),
                pl.BlockSpec((1, _CBI), lambda j, _i=iband: (0, _i)),
                pl.BlockSpec((1, _CBI), lambda j: (0, j)),
                pl.BlockSpec((_CBI, 1), lambda j, _i=iband: (_i, 0)),
            ],
            out_specs=[
                pl.BlockSpec((1, _CBI), lambda j: (0, j)),
                pl.BlockSpec((1, 1), lambda j: (0, 0)),
                pl.BlockSpec((1, 1), lambda j: (0, 0)),
            ],
            out_shape=[jax.ShapeDtypeStruct((1, NP), jnp.float32), sd, sd],
            scratch_shapes=[pltpu.VMEM((_CBI, 1), jnp.float32)],
        )(a, b, vmask_row, vmask_row, vmask)
        cs_bands.append(cs)
        d_parts.append(d[0, 0])
        lr_parts.append(lr[0, 0])
    cs_all = jnp.concatenate(cs_bands, axis=0)
    lc = pl.pallas_call(
        _colsum_fin_body,
        grid=(_CNB,),
        in_specs=[
            pl.BlockSpec((_CNB, _CBI), lambda j: (0, j)),
            pl.BlockSpec((1, _CBI), lambda j: (0, j)),
        ],
        out_specs=pl.BlockSpec((1, 1), lambda j: (0, 0)),
        out_shape=sd,
    )(cs_all, vmask_row)
    dsum = sum(d_parts)
    lrsum = sum(lr_parts)
    return -dsum / N + (lrsum + lc[0, 0]) / (2.0 * N)


def _rowatt_body(c0_ref, c1_ref, x_ref, p_ref, o_ref):
    p3 = c0_ref[...]
    p2 = c1_ref[...]
    avg0 = jnp.mean(p3, axis=1, keepdims=True)
    avg1 = jnp.mean(p2, axis=1, keepdims=True)
    aa = jnp.maximum(avg0 * p_ref[0, 0] + avg1 * p_ref[0, 1] + p_ref[0, 2], 0.0)
    at0 = jax.nn.sigmoid(aa * p_ref[0, 3] + p_ref[0, 5])
    at1 = jax.nn.sigmoid(aa * p_ref[0, 4] + p_ref[0, 6])
    o_ref[...] = (p3 * (at0 * p_ref[0, 7]) + p2 * (at1 * p_ref[0, 8])
                  + p_ref[0, 9] + x_ref[...])


def _rowatt(c0, c1, x, prm, br=2048):
    return pl.pallas_call(
        _rowatt_body,
        grid=(NP // br,),
        in_specs=[
            pl.BlockSpec((br, 128), lambda i: (i, 0)),
            pl.BlockSpec((br, 128), lambda i: (i, 0)),
            pl.BlockSpec((br, 128), lambda i: (i, 0)),
            pl.BlockSpec((1, 128), lambda i: (0, 0)),
        ],
        out_specs=pl.BlockSpec((br, 128), lambda i: (i, 0)),
        out_shape=jax.ShapeDtypeStruct((NP, 128), jnp.float32),
    )(c0, c1, x, prm)


def _mlp_body(g_ref, w1_ref, b1_ref, w2_ref, b2_ref, w3_ref, b3_ref, o_ref):
    h = (jnp.dot(g_ref[0], w1_ref[0], preferred_element_type=jnp.float32)
         + jnp.dot(g_ref[1], w1_ref[1], preferred_element_type=jnp.float32)
         + b1_ref[...])
    h = jnp.where(h > 0, h, 0.01 * h)
    h = jnp.dot(h, w2_ref[...], preferred_element_type=jnp.float32) + b2_ref[...]
    h = jnp.where(h > 0, h, 0.01 * h)
    o_ref[...] = (jnp.dot(h, w3_ref[...], preferred_element_type=jnp.float32)
                  + b3_ref[...])


def _mlp(g, w1, b1, w2, b2, w3, b3, br=1024):
    nb = 4096 // br
    return pl.pallas_call(
        _mlp_body,
        grid=(nb,),
        in_specs=[
            pl.BlockSpec((2, br, 128), lambda i: (0, i, 0)),
            pl.BlockSpec((2, 128, 64), lambda i: (0, 0, 0)),
            pl.BlockSpec((1, 64), lambda i: (0, 0)),
            pl.BlockSpec((64, 32), lambda i: (0, 0)),
            pl.BlockSpec((1, 32), lambda i: (0, 0)),
            pl.BlockSpec((32, 1), lambda i: (0, 0)),
            pl.BlockSpec((1, 1), lambda i: (0, 0)),
        ],
        out_specs=pl.BlockSpec((br, 1), lambda i: (i, 0)),
        out_shape=jax.ShapeDtypeStruct((4096, 1), jnp.float32),
    )(g, w1, b1, w2, b2, w3, b3)


def _tiled(x):
    return x.reshape(NTILES, NCHUNK, CHUNK)


def _he_plan(he):
    ni, ei = he[0], he[1]
    return {
        "nhalf": ni.reshape(NCORES, NTILES, SCH // SSUP, SSUP, SCHUNK),
        "ehalf": ei.reshape(NCORES, NTILES, SCH // SSUP, SSUP, SCHUNK),
    }


_BN_S = 1.0 / np.sqrt(1.0 + BN_EPS)


def _conv_slab(table, plan, binv_col, dinv_col, bias_row, zb128):
    pa = _sc_seghalf(table, plan["nhalf"], plan["ehalf"])
    ef = _merge(pa.reshape(2, NP, 128), binv_col, zb128)
    pb = _sc_seghalf(ef, plan["ehalf"], plan["nhalf"])
    return _merge(pb.reshape(2, NP, 128), dinv_col, bias_row)


def _run_hgnn(x_pad, plan, pp, binv_col, dinv_col):
    zb128 = jnp.zeros((1, 128), jnp.float32)
    w1 = pp["W1"].reshape(1, 128, 2, 128)
    b0 = jnp.zeros((2, 1, 128), jnp.float32)
    xl1 = _dense([x_pad], w1, b0)
    h1 = [
        _conv_slab(xl1[h], plan, binv_col, dinv_col,
                   pp["b1"][h * 128:(h + 1) * 128].reshape(1, 128), zb128)
        for h in range(2)
    ]
    s1 = pp["g1"] * _BN_S
    w2p = (s1[:, None] * pp["W2"]).reshape(2, 128, 1, 128)
    b2p = (pp["be1"] @ pp["W2"]).reshape(1, 1, 128)
    xl2 = _dense(h1, w2p, b2p)
    return _conv_slab(xl2[0], plan, binv_col, dinv_col,
                      pp["b2"].reshape(1, 128), zb128)


def _att_prm(ap):
    v = jnp.stack([ap["fc1W"][0, 0], ap["fc1W"][1, 0], ap["fc1b"][0],
                   ap["fc2W"][0, 0], ap["fc2W"][0, 1],
                   ap["fc2b"][0], ap["fc2b"][1],
                   ap["convW"][0], ap["convW"][1], ap["convb"][0]])
    return jnp.zeros((1, 128), jnp.float32).at[0, :10].set(v)


def kernel(x_protein, hyperedge_protein_index, x_meta, hyperedge_meta_index,
           index, params):
    he_p, he_m = hyperedge_protein_index, hyperedge_meta_index
    idx8 = jnp.stack([
        _tiled(he_p[0, 0]), _tiled(he_p[0, 1]),
        _tiled(he_p[1, 0]), _tiled(he_p[1, 1]),
        _tiled(he_m[0, 0]), _tiled(he_m[0, 1]),
        _tiled(he_m[1, 0]), _tiled(he_m[1, 1]),
    ], axis=0)
    inv = _sc_degrees(idx8)

    xp_pad = jnp.pad(x_protein, ((0, NP - N), (0, 0)))
    xm_pad = jnp.pad(x_meta, ((0, NP - N), (0, 0)))

    plans = [_he_plan(he_p[0]), _he_plan(he_p[1]),
             _he_plan(he_m[0]), _he_plan(he_m[1])]

    def col(j):
        return inv[j].reshape(NP, 1)

    p2 = _run_hgnn(xp_pad, plans[0], params["pc1"], col(1), col(0))
    p3 = _run_hgnn(xp_pad, plans[1], params["pc2"], col(3), col(2))
    m2 = _run_hgnn(xm_pad, plans[2], params["mc1"], col(5), col(4))
    m3 = _run_hgnn(xm_pad, plans[3], params["mc2"], col(7), col(6))

    vmask = (jnp.arange(NP) < N).astype(jnp.float32).reshape(NP, 1)
    loss_p = _contrast_tc(p2, p3, vmask)
    loss_m = _contrast_tc(m2, m3, vmask)
    loss = (jnp.exp(-params["p12"]) * loss_p + params["p12"]
            + jnp.exp(-params["m13"]) * loss_m + params["m13"])

    protein = _rowatt(p3, p2, xp_pad, _att_prm(params["attp"]))
    meta = _rowatt(m3, m2, xm_pad, _att_prm(params["attm"]))

    tables = jnp.concatenate([protein, meta], axis=0)
    idx_all = jnp.concatenate([index[0], index[1] + NP]).reshape(32, 2, 128)
    g = _sc_gather(tables, idx_all)

    e = params["enc"]
    s1 = e["g1"] * _BN_S
    w1 = (e["W1"] * s1[None, :]).reshape(2, 128, 64)
    b1 = (e["b1"] * s1 + e["be1"]).reshape(1, 64)
    s2 = e["g2"] * _BN_S
    w2 = e["W2"] * s2[None, :]
    b2 = (e["b2"] * s2 + e["be2"]).reshape(1, 32)
    out = _mlp(g.reshape(2, 4096, 128), w1, b1, w2, b2,
               e["W3"], e["b3"].reshape(1, 1))
    return out, loss


# speedup vs baseline: 5.1981x; 1.0004x over previous
import functools

import jax
import jax.numpy as jnp
import numpy as np
from jax import lax
from jax.experimental import pallas as pl
from jax.experimental.pallas import tpu as pltpu
from jax.experimental.pallas import tpu_sc as plsc

N = 10000
NP = 10240
E = 160000
BN_EPS = 1e-5
NTILES = 16
NCORES = 2
CHUNK = 125
NCHUNK = E // NTILES // CHUNK
SCHUNK = 100
SCH = E // NCORES // NTILES // SCHUNK
SSUP = 25
RPT = NP // NTILES
DH2 = 128


@functools.lru_cache(maxsize=None)
def _mesh():
    return plsc.VectorSubcoreMesh(core_axis_name="c", subcore_axis_name="s")


@functools.lru_cache(maxsize=None)
def _make_degrees():
    return functools.partial(
        pl.kernel,
        mesh=_mesh(),
        out_type=jax.ShapeDtypeStruct((8, NP), jnp.float32),
        scratch_types=[
            pltpu.VMEM((NCHUNK, CHUNK), jnp.int32),
            pltpu.VMEM((CHUNK,), jnp.float32),
            pltpu.VMEM((RPT,), jnp.float32),
            pltpu.VMEM_SHARED((NP,), jnp.float32),
            pltpu.SemaphoreType.DMA,
        ],
    )(_sc_degrees_body)


def _sc_degrees_body(idx_hbm, ones_hbm, zeros_hbm, inv_hbm,
                     idx_v, ones_v, val_v, acc, sem):
    c = lax.axis_index("c")
    s = lax.axis_index("s")
    pltpu.sync_copy(ones_hbm, ones_v)
    for jb in range(4):
        job = c * 4 + jb
        pltpu.sync_copy(zeros_hbm, acc.at[pl.ds(s * RPT, RPT)])
        pltpu.sync_copy(idx_hbm.at[job, s], idx_v)
        plsc.subcore_barrier()

        def grp(g, _):
            descs = []
            for u in range(8):
                descs.append(pltpu.async_copy(
                    ones_v, acc.at[idx_v.at[g * 8 + u]], sem, add=True))
            for d in descs:
                d.wait()
            return ()

        lax.fori_loop(0, NCHUNK // 8, grp, ())
        plsc.subcore_barrier()
        pltpu.sync_copy(acc.at[pl.ds(s * RPT, RPT)], val_v)
        for l in range(RPT // 16):
            sl = pl.ds(l * 16, 16)
            v = val_v[sl]
            val_v[sl] = jnp.where(v > 0.0, 1.0 / v, 0.0)
        pltpu.sync_copy(val_v, inv_hbm.at[job, pl.ds(s * RPT, RPT)])
        plsc.subcore_barrier()


def _sc_degrees(idx8):
    ones = jnp.ones((CHUNK,), jnp.float32)
    zeros = jnp.zeros((RPT,), jnp.float32)
    return _make_degrees()(idx8, ones, zeros)


@functools.lru_cache(maxsize=None)
def _make_seghalf():
    @functools.partial(
        pl.kernel,
        mesh=_mesh(),
        out_type=jax.ShapeDtypeStruct((NCORES * NP, DH2), jnp.float32),
        scratch_types=[
            pltpu.VMEM((SSUP, SCHUNK), jnp.int32),
            pltpu.VMEM((SSUP, SCHUNK), jnp.int32),
            pltpu.VMEM((SCHUNK, DH2), jnp.float32),
            pltpu.VMEM((SCHUNK, DH2), jnp.float32),
            pltpu.VMEM((SCHUNK, DH2), jnp.float32),
            pltpu.VMEM_SHARED((NP, DH2), jnp.float32),
            pltpu.SemaphoreType.DMA,
            pltpu.SemaphoreType.DMA,
        ],
    )
    def seghalf(table, src, dst, zeros, out,
                src_v, dst_v, buf0, buf1, buf2, acc, gsem, ssem):
        c = lax.axis_index("c")
        s = lax.axis_index("s")
        bufs = (buf0, buf1, buf2)

        def gwait(j, b):
            pltpu.make_async_copy(
                table.at[src_v.at[j]], bufs[b], gsem).wait()

        def swait(j, b):
            pltpu.make_async_copy(
                bufs[b], acc.at[dst_v.at[j]], ssem).wait()

        pltpu.sync_copy(zeros, acc.at[pl.ds(s * RPT, RPT)])
        plsc.subcore_barrier()

        for si in range(SCH // SSUP):
            pltpu.sync_copy(src.at[c, s, si], src_v)
            pltpu.sync_copy(dst.at[c, s, si], dst_v)
            pltpu.async_copy(table.at[src_v.at[0]], buf0, gsem)
            pltpu.async_copy(table.at[src_v.at[1]], buf1, gsem)

            def step(j3, _):
                for u in range(3):
                    l = j3 * 3 + u
                    gwait(l, u)
                    pltpu.async_copy(bufs[u], acc.at[dst_v.at[l]], ssem,
                                     add=True)

                    @pl.when(l >= 1)
                    def _():
                        swait(l - 1, (u + 2) % 3)

                    @pl.when(l + 2 < SSUP)
                    def _():
                        pltpu.async_copy(
                            table.at[src_v.at[l + 2]], bufs[(u + 2) % 3],
                            gsem)
                return ()

            lax.fori_loop(0, (SSUP - 1) // 3, step, ())
            l = SSUP - 1
            gwait(l, l % 3)
            pltpu.async_copy(bufs[l % 3], acc.at[dst_v.at[l]], ssem, add=True)
            swait(l - 1, (l - 1) % 3)
            swait(l, l % 3)
        plsc.subcore_barrier()
        pltpu.sync_copy(acc.at[pl.ds(s * RPT, RPT)],
                        out.at[pl.ds(c * NP + s * RPT, RPT)])

    return seghalf


def _sc_seghalf(table, src, dst):
    zeros = jnp.zeros((RPT, DH2), jnp.float32)
    return _make_seghalf()(table, src, dst, zeros)


@functools.lru_cache(maxsize=None)
def _make_gather():
    return functools.partial(
        pl.kernel,
        mesh=_mesh(),
        out_type=jax.ShapeDtypeStruct((8192, 128), jnp.float32),
        scratch_types=[
            pltpu.VMEM((2, 128), jnp.int32),
            pltpu.VMEM((128, 128), jnp.float32),
            pltpu.VMEM((128, 128), jnp.float32),
            pltpu.SemaphoreType.DMA,
        ],
    )(_sc_gather_body)


def _sc_gather(tables, idx_all):
    return _make_gather()(tables, idx_all)


def _sc_gather_body(tab, idx, out, idx_v, bufa, bufb, sem):
    c = lax.axis_index("c")
    s = lax.axis_index("s")
    w = s * NCORES + c
    pltpu.sync_copy(idx.at[w], idx_v)
    da = pltpu.async_copy(tab.at[idx_v.at[0]], bufa, sem)
    db = pltpu.async_copy(tab.at[idx_v.at[1]], bufb, sem)
    da.wait()
    pltpu.sync_copy(bufa, out.at[pl.ds(w * 256, 128)])
    db.wait()
    pltpu.sync_copy(bufb, out.at[pl.ds(w * 256 + 128, 128)])


def _merge_body(p_ref, rs_ref, b_ref, o_ref):
    o_ref[...] = (p_ref[0] + p_ref[1]) * rs_ref[...] + b_ref[...]


def _merge(parts, rowscale, bias, br=2048):
    return pl.pallas_call(
        _merge_body,
        grid=(NP // br,),
        in_specs=[
            pl.BlockSpec((2, br, 128), lambda i: (0, i, 0)),
            pl.BlockSpec((br, 1), lambda i: (i, 0)),
            pl.BlockSpec((1, 128), lambda i: (0, 0)),
        ],
        out_specs=pl.BlockSpec((br, 128), lambda i: (i, 0)),
        out_shape=jax.ShapeDtypeStruct((NP, 128), jnp.float32),
    )(parts, rowscale, bias)


def _dense(xs, w, b, br=2048):
    kc = len(xs)
    n, k2 = xs[0].shape
    ns, nh2 = w.shape[2], w.shape[3]

    def body(*refs):
        x_refs = refs[:kc]
        w_ref, b_ref, o_ref = refs[kc], refs[kc + 1], refs[kc + 2]
        for h in range(ns):
            acc = jnp.dot(x_refs[0][...], w_ref[0, :, h],
                          preferred_element_type=jnp.float32)
            for k in range(1, kc):
                acc += jnp.dot(x_refs[k][...], w_ref[k, :, h],
                               preferred_element_type=jnp.float32)
            o_ref[h] = acc + b_ref[h]

    return pl.pallas_call(
        body,
        grid=(n // br,),
        in_specs=[pl.BlockSpec((br, k2), lambda i: (i, 0))] * kc + [
            pl.BlockSpec((kc, k2, ns, nh2), lambda i: (0, 0, 0, 0)),
            pl.BlockSpec((ns, 1, nh2), lambda i: (0, 0, 0)),
        ],
        out_specs=pl.BlockSpec((ns, br, nh2), lambda i: (0, i, 0)),
        out_shape=jax.ShapeDtypeStruct((ns, n, nh2), jnp.float32),
    )(*xs, w, b)


_CBI = 1024
_CNB = NP // _CBI


def _make_band_body(iband):
    def body(a_ref, b_ref, rm_ref, cm_ref, rmc_ref, cs_ref, d_ref, lr_ref,
             rowacc):
        j = pl.program_id(0)
        a = a_ref[...]
        b = b_ref[...]
        a = a * lax.rsqrt(jnp.maximum(jnp.sum(a * a, 1, keepdims=True),
                                      1e-24))
        b = b * lax.rsqrt(jnp.maximum(jnp.sum(b * b, 1, keepdims=True),
                                      1e-24))
        s = lax.dot_general(a.astype(jnp.bfloat16), b.astype(jnp.bfloat16),
                            (((1,), (1,)), ((), ())),
                            preferred_element_type=jnp.float32) * (1.0 / 0.7)
        ex = jnp.exp(s) * cm_ref[...]
        rs_ = jnp.sum(ex, axis=1, keepdims=True)
        cs_ref[...] = jnp.sum(ex * rm_ref[...], axis=0, keepdims=True)

        @pl.when(j == 0)
        def _():
            rowacc[...] = rs_
            d_ref[...] = jnp.zeros((1, 1), jnp.float32)

        @pl.when(j > 0)
        def _():
            rowacc[...] += rs_

        @pl.when(j == iband)
        def _():
            rid = lax.broadcasted_iota(jnp.int32, (_CBI, _CBI), 0)
            cid = lax.broadcasted_iota(jnp.int32, (_CBI, _CBI), 1)
            gid = iband * _CBI + rid
            d_ref[...] += jnp.sum(
                jnp.where((rid == cid) & (gid < N), s, 0.0)).reshape(1, 1)

        @pl.when(j == _CNB - 1)
        def _():
            lr_ref[...] = jnp.sum(
                rmc_ref[...] * jnp.log(rowacc[...])).reshape(1, 1)

    return body


def _colsum_fin_body(cs_ref, m_ref, o_ref):
    tot = cs_ref[0]
    for k in range(1, _CNB):
        tot = tot + cs_ref[k]
    j = pl.program_id(0)
    part = jnp.sum(jnp.where(m_ref[...] > 0.0, jnp.log(tot),
                             0.0)).reshape(1, 1)

    @pl.when(j == 0)
    def _():
        o_ref[...] = part

    @pl.when(j > 0)
    def _():
        o_ref[...] += part


def _contrast_tc(a, b, vmask):
    sd = jax.ShapeDtypeStruct((1, 1), jnp.float32)
    vmask_row = vmask.reshape(1, NP)
    cs_bands, d_parts, lr_parts = [], [], []
    for iband in range(_CNB):
        cs, d, lr = pl.pallas_call(
            _make_band_body(iband),
            grid=(_CNB,),
            in_specs=[
                pl.BlockSpec((_CBI, 128), lambda j, _i=iband: (_i, 0)),
                pl.BlockSpec((_CBI, 128), lambda j: (j, 0)),
                pl.BlockSpec((1, _CBI), lambda j, _i=iband: (0, _i)),
                pl.BlockSpec((1, _CBI), lambda j: (0, j)),
                pl.BlockSpec((_CBI, 1), lambda j, _i=iband: (_i, 0)),
            ],
            out_specs=[
                pl.BlockSpec((1, _CBI), lambda j: (0, j)),
                pl.BlockSpec((1, 1), lambda j: (0, 0)),
                pl.BlockSpec((1, 1), lambda j: (0, 0)),
            ],
            out_shape=[jax.ShapeDtypeStruct((1, NP), jnp.float32), sd, sd],
            scratch_shapes=[pltpu.VMEM((_CBI, 1), jnp.float32)],
        )(a, b, vmask_row, vmask_row, vmask)
        cs_bands.append(cs)
        d_parts.append(d[0, 0])
        lr_parts.append(lr[0, 0])
    cs_all = jnp.concatenate(cs_bands, axis=0)
    lc = pl.pallas_call(
        _colsum_fin_body,
        grid=(_CNB,),
        in_specs=[
            pl.BlockSpec((_CNB, _CBI), lambda j: (0, j)),
            pl.BlockSpec((1, _CBI), lambda j: (0, j)),
        ],
        out_specs=pl.BlockSpec((1, 1), lambda j: (0, 0)),
        out_shape=sd,
    )(cs_all, vmask_row)
    dsum = sum(d_parts)
    lrsum = sum(lr_parts)
    return -dsum / N + (lrsum + lc[0, 0]) / (2.0 * N)


def _rowatt_body(c0_ref, c1_ref, x_ref, p_ref, o_ref):
    p3 = c0_ref[...]
    p2 = c1_ref[...]
    avg0 = jnp.mean(p3, axis=1, keepdims=True)
    avg1 = jnp.mean(p2, axis=1, keepdims=True)
    aa = jnp.maximum(avg0 * p_ref[0, 0] + avg1 * p_ref[0, 1] + p_ref[0, 2], 0.0)
    at0 = jax.nn.sigmoid(aa * p_ref[0, 3] + p_ref[0, 5])
    at1 = jax.nn.sigmoid(aa * p_ref[0, 4] + p_ref[0, 6])
    o_ref[...] = (p3 * (at0 * p_ref[0, 7]) + p2 * (at1 * p_ref[0, 8])
                  + p_ref[0, 9] + x_ref[...])


def _rowatt(c0, c1, x, prm, br=2048):
    return pl.pallas_call(
        _rowatt_body,
        grid=(NP // br,),
        in_specs=[
            pl.BlockSpec((br, 128), lambda i: (i, 0)),
            pl.BlockSpec((br, 128), lambda i: (i, 0)),
            pl.BlockSpec((br, 128), lambda i: (i, 0)),
            pl.BlockSpec((1, 128), lambda i: (0, 0)),
        ],
        out_specs=pl.BlockSpec((br, 128), lambda i: (i, 0)),
        out_shape=jax.ShapeDtypeStruct((NP, 128), jnp.float32),
    )(c0, c1, x, prm)


def _mlp_body(g_ref, w1_ref, b1_ref, w2_ref, b2_ref, w3_ref, b3_ref, o_ref):
    h = (jnp.dot(g_ref[0], w1_ref[0], preferred_element_type=jnp.float32)
         + jnp.dot(g_ref[1], w1_ref[1], preferred_element_type=jnp.float32)
         + b1_ref[...])
    h = jnp.where(h > 0, h, 0.01 * h)
    h = jnp.dot(h, w2_ref[...], preferred_element_type=jnp.float32) + b2_ref[...]
    h = jnp.where(h > 0, h, 0.01 * h)
    o_ref[...] = (jnp.dot(h, w3_ref[...], preferred_element_type=jnp.float32)
                  + b3_ref[...])


def _mlp(g, w1, b1, w2, b2, w3, b3, br=1024):
    nb = 4096 // br
    return pl.pallas_call(
        _mlp_body,
        grid=(nb,),
        in_specs=[
            pl.BlockSpec((2, br, 128), lambda i: (0, i, 0)),
            pl.BlockSpec((2, 128, 64), lambda i: (0, 0, 0)),
            pl.BlockSpec((1, 64), lambda i: (0, 0)),
            pl.BlockSpec((64, 32), lambda i: (0, 0)),
            pl.BlockSpec((1, 32), lambda i: (0, 0)),
            pl.BlockSpec((32, 1), lambda i: (0, 0)),
            pl.BlockSpec((1, 1), lambda i: (0, 0)),
        ],
        out_specs=pl.BlockSpec((br, 1), lambda i: (i, 0)),
        out_shape=jax.ShapeDtypeStruct((4096, 1), jnp.float32),
    )(g, w1, b1, w2, b2, w3, b3)


def _tiled(x):
    return x.reshape(NTILES, NCHUNK, CHUNK)


def _he_plan(he):
    ni, ei = he[0], he[1]
    return {
        "nhalf": ni.reshape(NCORES, NTILES, SCH // SSUP, SSUP, SCHUNK),
        "ehalf": ei.reshape(NCORES, NTILES, SCH // SSUP, SSUP, SCHUNK),
    }


_BN_S = 1.0 / np.sqrt(1.0 + BN_EPS)


def _conv_slab(table, plan, binv_col, dinv_col, bias_row, zb128):
    pa = _sc_seghalf(table, plan["nhalf"], plan["ehalf"])
    ef = _merge(pa.reshape(2, NP, 128), binv_col, zb128)
    pb = _sc_seghalf(ef, plan["ehalf"], plan["nhalf"])
    return _merge(pb.reshape(2, NP, 128), dinv_col, bias_row)


def _run_hgnn(x_pad, plan, pp, binv_col, dinv_col):
    zb128 = jnp.zeros((1, 128), jnp.float32)
    w1 = pp["W1"].reshape(1, 128, 2, 128)
    b0 = jnp.zeros((2, 1, 128), jnp.float32)
    xl1 = _dense([x_pad], w1, b0)
    h1 = [
        _conv_slab(xl1[h], plan, binv_col, dinv_col,
                   pp["b1"][h * 128:(h + 1) * 128].reshape(1, 128), zb128)
        for h in range(2)
    ]
    s1 = pp["g1"] * _BN_S
    w2p = (s1[:, None] * pp["W2"]).reshape(2, 128, 1, 128)
    b2p = (pp["be1"] @ pp["W2"]).reshape(1, 1, 128)
    xl2 = _dense(h1, w2p, b2p)
    return _conv_slab(xl2[0], plan, binv_col, dinv_col,
                      pp["b2"].reshape(1, 128), zb128)


def _att_prm(ap):
    v = jnp.stack([ap["fc1W"][0, 0], ap["fc1W"][1, 0], ap["fc1b"][0],
                   ap["fc2W"][0, 0], ap["fc2W"][0, 1],
                   ap["fc2b"][0], ap["fc2b"][1],
                   ap["convW"][0], ap["convW"][1], ap["convb"][0]])
    return jnp.zeros((1, 128), jnp.float32).at[0, :10].set(v)


def kernel(x_protein, hyperedge_protein_index, x_meta, hyperedge_meta_index,
           index, params):
    he_p, he_m = hyperedge_protein_index, hyperedge_meta_index
    idx8 = jnp.stack([
        _tiled(he_p[0, 0]), _tiled(he_p[0, 1]),
        _tiled(he_p[1, 0]), _tiled(he_p[1, 1]),
        _tiled(he_m[0, 0]), _tiled(he_m[0, 1]),
        _tiled(he_m[1, 0]), _tiled(he_m[1, 1]),
    ], axis=0)
    inv = _sc_degrees(idx8)

    xp_pad = jnp.pad(x_protein, ((0, NP - N), (0, 0)))
    xm_pad = jnp.pad(x_meta, ((0, NP - N), (0, 0)))

    plans = [_he_plan(he_p[0]), _he_plan(he_p[1]),
             _he_plan(he_m[0]), _he_plan(he_m[1])]

    def col(j):
        return inv[j].reshape(NP, 1)

    p2 = _run_hgnn(xp_pad, plans[0], params["pc1"], col(1), col(0))
    p3 = _run_hgnn(xp_pad, plans[1], params["pc2"], col(3), col(2))
    m2 = _run_hgnn(xm_pad, plans[2], params["mc1"], col(5), col(4))
    m3 = _run_hgnn(xm_pad, plans[3], params["mc2"], col(7), col(6))

    protein = _rowatt(p3, p2, xp_pad, _att_prm(params["attp"]))
    meta = _rowatt(m3, m2, xm_pad, _att_prm(params["attm"]))

    tables = jnp.concatenate([protein, meta], axis=0)
    idx_all = jnp.concatenate([index[0], index[1] + NP]).reshape(32, 2, 128)
    g = _sc_gather(tables, idx_all)

    vmask = (jnp.arange(NP) < N).astype(jnp.float32).reshape(NP, 1)
    loss_p = _contrast_tc(p2, p3, vmask)
    loss_m = _contrast_tc(m2, m3, vmask)
    loss = (jnp.exp(-params["p12"]) * loss_p + params["p12"]
            + jnp.exp(-params["m13"]) * loss_m + params["m13"])

    e = params["enc"]
    s1 = e["g1"] * _BN_S
    w1 = (e["W1"] * s1[None, :]).reshape(2, 128, 64)
    b1 = (e["b1"] * s1 + e["be1"]).reshape(1, 64)
    s2 = e["g2"] * _BN_S
    w2 = e["W2"] * s2[None, :]
    b2 = (e["b2"] * s2 + e["be2"]).reshape(1, 32)
    out = _mlp(g.reshape(2, 4096, 128), w1, b1, w2, b2,
               e["W3"], e["b3"].reshape(1, 1))
    return out, loss
